# Initial kernel scaffold; baseline (speedup 1.0000x reference)
import functools

import jax
import jax.numpy as jnp
from jax import lax
from jax.experimental import pallas as pl
from jax.experimental.pallas import tpu as pltpu
from jax.experimental.pallas import tpu_sc as plsc

N_NODES = 10000
N_EDGES_IN = 320000
N_E = N_EDGES_IN + N_NODES
D_FEAT = 128
HEADS1 = 8
C1 = 16
C2 = 64

N_PAD = 10240
K = 512
KT = 128
E_PAD = 344064
ROWS_PER_TILE = N_PAD // 16

TCB = 2048


def _tc_proj_body(x_ref, w_ref, ss_ref, sd_ref, h_ref, as_ref, ad_ref):
    h = jnp.dot(x_ref[...], w_ref[...], preferred_element_type=jnp.float32)
    h_ref[...] = h
    as_ref[...] = jnp.dot(h, ss_ref[...], preferred_element_type=jnp.float32)
    ad_ref[...] = jnp.dot(h, sd_ref[...], preferred_element_type=jnp.float32)


def _tc_proj(x, W, Ss, Sd):
    n, d = x.shape
    dc = W.shape[1]
    return pl.pallas_call(
        _tc_proj_body,
        grid=(n // TCB,),
        in_specs=[
            pl.BlockSpec((TCB, d), lambda i: (i, 0)),
            pl.BlockSpec((d, dc), lambda i: (0, 0)),
            pl.BlockSpec((dc, 8), lambda i: (0, 0)),
            pl.BlockSpec((dc, 8), lambda i: (0, 0)),
        ],
        out_specs=[
            pl.BlockSpec((TCB, dc), lambda i: (i, 0)),
            pl.BlockSpec((TCB, 8), lambda i: (i, 0)),
            pl.BlockSpec((TCB, 8), lambda i: (i, 0)),
        ],
        out_shape=[
            jax.ShapeDtypeStruct((n, dc), jnp.float32),
            jax.ShapeDtypeStruct((n, 8), jnp.float32),
            jax.ShapeDtypeStruct((n, 8), jnp.float32),
        ],
    )(x, W, Ss, Sd)


def _tc_mid_body(p0_ref, p1_ref, b_ref, w_ref, ss_ref, sd_ref,
                 h_ref, as_ref, ad_ref):
    hp = p0_ref[...] + p1_ref[...] + b_ref[...]
    hp = jnp.where(hp > 0, hp, jnp.expm1(hp))
    h = jnp.dot(hp, w_ref[...], preferred_element_type=jnp.float32)
    h_ref[...] = h
    as_ref[...] = jnp.dot(h, ss_ref[...], preferred_element_type=jnp.float32)
    ad_ref[...] = jnp.dot(h, sd_ref[...], preferred_element_type=jnp.float32)


def _tc_mid(p0, p1, b1, W2, Ss, Sd):
    n, d = p0.shape
    dc = W2.shape[1]
    return pl.pallas_call(
        _tc_mid_body,
        grid=(n // TCB,),
        in_specs=[
            pl.BlockSpec((TCB, d), lambda i: (i, 0)),
            pl.BlockSpec((TCB, d), lambda i: (i, 0)),
            pl.BlockSpec((1, d), lambda i: (0, 0)),
            pl.BlockSpec((d, dc), lambda i: (0, 0)),
            pl.BlockSpec((dc, 8), lambda i: (0, 0)),
            pl.BlockSpec((dc, 8), lambda i: (0, 0)),
        ],
        out_specs=[
            pl.BlockSpec((TCB, dc), lambda i: (i, 0)),
            pl.BlockSpec((TCB, 8), lambda i: (i, 0)),
            pl.BlockSpec((TCB, 8), lambda i: (i, 0)),
        ],
        out_shape=[
            jax.ShapeDtypeStruct((n, dc), jnp.float32),
            jax.ShapeDtypeStruct((n, 8), jnp.float32),
            jax.ShapeDtypeStruct((n, 8), jnp.float32),
        ],
    )(p0, p1, b1, W2, Ss, Sd)


def _tc_final_body(p0_ref, p1_ref, b_ref, o_ref):
    o_ref[...] = p0_ref[...] + p1_ref[...] + b_ref[...]


def _tc_final(p0, p1, b2):
    n, d = p0.shape
    return pl.pallas_call(
        _tc_final_body,
        grid=(n // TCB,),
        in_specs=[
            pl.BlockSpec((TCB, d), lambda i: (i, 0)),
            pl.BlockSpec((TCB, d), lambda i: (i, 0)),
            pl.BlockSpec((1, d), lambda i: (0, 0)),
        ],
        out_specs=pl.BlockSpec((TCB, d), lambda i: (i, 0)),
        out_shape=jax.ShapeDtypeStruct((n, d), jnp.float32),
    )(p0, p1, b2)


def _make_sc_gat(HC, C):
    NV = HC // 16
    E1_TILE = E_PAD // 16
    E2_TILE = E_PAD // 32
    NCH1 = E1_TILE // K
    NCH2 = E2_TILE // K
    NZB = ROWS_PER_TILE // KT

    mesh = plsc.VectorSubcoreMesh(core_axis_name="c", subcore_axis_name="s")

    @functools.partial(
        pl.kernel,
        out_type=[
            jax.ShapeDtypeStruct((2, N_PAD, HC), jnp.float32),
            jax.ShapeDtypeStruct((E_PAD, 8), jnp.float32),
        ],
        mesh=mesh,
        scratch_types=[
            pltpu.VMEM_SHARED((N_PAD, 8), jnp.float32),
            pltpu.VMEM_SHARED((N_PAD, 8), jnp.float32),
            pltpu.VMEM_SHARED((N_PAD, 8), jnp.float32),
            pltpu.VMEM_SHARED((N_PAD, HC), jnp.float32),
            pltpu.VMEM((4, KT), jnp.int32),
            pltpu.VMEM((4, KT), jnp.int32),
            pltpu.VMEM((K, 8), jnp.float32),
            pltpu.VMEM((K, 8), jnp.float32),
            pltpu.VMEM((K, 8), jnp.float32),
            pltpu.VMEM((K * 8,), jnp.float32),
            pltpu.VMEM((K, HC), jnp.float32),
            pltpu.VMEM((KT, 8), jnp.float32),
            pltpu.SemaphoreType.DMA,
        ],
    )
    def sc_gat(h_hbm, as_hbm, ad_hbm, src_hbm, dst_hbm, out_hbm, ee_hbm,
               AS, AD, DEN, OUT, srcb, dstb, asr, adr, eer, alphab, hrows,
               zb, sem):
        c = lax.axis_index("c")
        s = lax.axis_index("s")
        row0 = s * ROWS_PER_TILE
        iota = lax.iota(jnp.int32, 16)
        pr = iota >> 3
        pc = iota & 7
        zv = jnp.zeros((16,), jnp.float32)

        def zrow(i, _):
            plsc.store_scatter(zb, [pr + i * 2, pc], zv)
            return 0
        lax.fori_loop(0, KT // 2, zrow, 0)

        for j in range(NZB):
            r = row0 + j * KT
            pltpu.sync_copy(as_hbm.at[pl.ds(r, KT)], AS.at[pl.ds(r, KT)])
            pltpu.sync_copy(ad_hbm.at[pl.ds(r, KT)], AD.at[pl.ds(r, KT)])
            pltpu.sync_copy(zb, DEN.at[pl.ds(r, KT)])
            for q in range(HC // 8):
                pltpu.sync_copy(zb, OUT.at[pl.ds(r, KT), pl.ds(q * 8, 8)])
        plsc.subcore_barrier()

        base1 = s * E1_TILE
        write_ee = (s >> 3) == c

        def p1_chunk(ch, _):
            off = base1 + ch * K
            r0 = off // KT
            pltpu.sync_copy(src_hbm.at[pl.ds(r0, 4)], srcb)
            pltpu.sync_copy(dst_hbm.at[pl.ds(r0, 4)], dstb)
            ds = []
            for j in range(4):
                ds.append(pltpu.async_copy(
                    AS.at[srcb.at[j]], asr.at[pl.ds(j * KT, KT)], sem))
                ds.append(pltpu.async_copy(
                    AD.at[dstb.at[j]], adr.at[pl.ds(j * KT, KT)], sem))
            for d in ds:
                d.wait()

            def body(i, _):
                k = i * 2
                a = plsc.load_gather(asr, [pr + k, pc])
                b = plsc.load_gather(adr, [pr + k, pc])
                e = a + b
                e = jnp.maximum(e, 0.2 * e)
                plsc.store_scatter(eer, [pr + k, pc], jnp.exp(e))
                return 0
            lax.fori_loop(0, K // 2, body, 0)

            for j in range(4):
                pltpu.sync_copy(eer.at[pl.ds(j * KT, KT)],
                                DEN.at[dstb.at[j]], add=True)

            @pl.when(write_ee)
            def _():
                pltpu.sync_copy(eer, ee_hbm.at[pl.ds(off, K)])
            return 0
        lax.fori_loop(0, NCH1, p1_chunk, 0)
        plsc.subcore_barrier()

        base2 = c * (E_PAD // 2) + s * E2_TILE

        def p2_chunk(ch, _):
            off = base2 + ch * K
            r0 = off // KT
            pltpu.sync_copy(src_hbm.at[pl.ds(r0, 4)], srcb)
            pltpu.sync_copy(dst_hbm.at[pl.ds(r0, 4)], dstb)
            pltpu.sync_copy(ee_hbm.at[pl.ds(off, K)], eer)
            ds = []
            for j in range(4):
                ds.append(pltpu.async_copy(
                    h_hbm.at[srcb.at[j]], hrows.at[pl.ds(j * KT, KT)], sem))
                ds.append(pltpu.async_copy(
                    DEN.at[dstb.at[j]], asr.at[pl.ds(j * KT, KT)], sem))
            for d in ds:
                d.wait()

            def abody(i, _):
                k = i * 2
                ee = plsc.load_gather(eer, [pr + k, pc])
                dn = plsc.load_gather(asr, [pr + k, pc])
                alphab[pl.ds(i * 16, 16)] = ee / (dn + 1e-16)
                return 0
            lax.fori_loop(0, K // 2, abody, 0)

            def mbody(k, _):
                k8 = k * 8
                for v in range(NV):
                    av = plsc.load_gather(
                        alphab,
                        [jnp.full((16,), k8 + (v * 16) // C, jnp.int32)])
                    hv = hrows[k, pl.ds(v * 16, 16)]
                    hrows[k, pl.ds(v * 16, 16)] = hv * av
                return 0
            lax.fori_loop(0, K, mbody, 0)

            for j in range(4):
                pltpu.sync_copy(hrows.at[pl.ds(j * KT, KT)],
                                OUT.at[dstb.at[j]], add=True)
            return 0
        lax.fori_loop(0, NCH2, p2_chunk, 0)
        plsc.subcore_barrier()

        for j in range(NZB):
            r = row0 + j * KT
            pltpu.sync_copy(OUT.at[pl.ds(r, KT)], out_hbm.at[c, pl.ds(r, KT)])

    return sc_gat


_sc_gat1 = _make_sc_gat(HEADS1 * C1, C1)
_sc_gat2 = _make_sc_gat(C2, C2)


def _att_mat(att, dc):
    H, _ = att.shape
    S = att[:, :, None] * jnp.eye(H, dtype=jnp.float32)[:, None, :]
    S = S.reshape(dc, H)
    return jnp.pad(S, ((0, 0), (0, 8 - H)))


def kernel(x, edge_index, W1, att_src1, att_dst1, b1, W2, att_src2, att_dst2,
           b2):
    xp = jnp.pad(x, ((0, N_PAD - N_NODES), (0, 0)))
    loops = jnp.arange(N_NODES, dtype=jnp.int32)
    fill = jnp.full((E_PAD - N_E,), N_NODES, jnp.int32)
    src = jnp.concatenate([edge_index[0].astype(jnp.int32), loops, fill])
    dst = jnp.concatenate([edge_index[1].astype(jnp.int32), loops, fill])
    src2d = src.reshape(E_PAD // KT, KT)
    dst2d = dst.reshape(E_PAD // KT, KT)

    Ss1 = _att_mat(att_src1, D_FEAT)
    Sd1 = _att_mat(att_dst1, D_FEAT)
    Ss2 = _att_mat(att_src2, C2)
    Sd2 = _att_mat(att_dst2, C2)

    h1, AS1, AD1 = _tc_proj(xp, W1, Ss1, Sd1)
    p1, _ = _sc_gat1(h1, AS1, AD1, src2d, dst2d)
    h2, AS2, AD2 = _tc_mid(p1[0], p1[1], b1.reshape(1, -1), W2, Ss2, Sd2)
    p2, _ = _sc_gat2(h2, AS2, AD2, src2d, dst2d)
    out = _tc_final(p2[0], p2[1], b2.reshape(1, -1))
    return out[:N_NODES]


# speedup vs baseline: 13.7333x; 13.7333x over previous
import functools

import jax
import jax.numpy as jnp
from jax import lax
from jax.experimental import pallas as pl
from jax.experimental.pallas import tpu as pltpu
from jax.experimental.pallas import tpu_sc as plsc

N_NODES = 10000
N_EDGES_IN = 320000
N_E = N_EDGES_IN + N_NODES
D_FEAT = 128
HEADS1 = 8
C1 = 16
C2 = 64

N_PAD = 10112
K = 128
E_PAD = 331776
ROWS_PER_TILE = N_PAD // 16
W_TILE = N_PAD * 8 // 16

TCB = 632


def _chunks(total, step):
    out = []
    o = 0
    while o < total:
        out.append((o, min(step, total - o)))
        o += step
    return out


def _tc_proj_body(x_ref, w_ref, ss_ref, sd_ref, h_ref, as_ref, ad_ref):
    h = jnp.dot(x_ref[...], w_ref[...], preferred_element_type=jnp.float32)
    h_ref[...] = h
    as_ref[...] = jnp.dot(h, ss_ref[...], preferred_element_type=jnp.float32)
    ad_ref[...] = jnp.dot(h, sd_ref[...], preferred_element_type=jnp.float32)


def _tc_proj(x, W, Ss, Sd):
    n, d = x.shape
    dc = W.shape[1]
    return pl.pallas_call(
        _tc_proj_body,
        grid=(n // TCB,),
        in_specs=[
            pl.BlockSpec((TCB, d), lambda i: (i, 0)),
            pl.BlockSpec((d, dc), lambda i: (0, 0)),
            pl.BlockSpec((dc, 8), lambda i: (0, 0)),
            pl.BlockSpec((dc, 8), lambda i: (0, 0)),
        ],
        out_specs=[
            pl.BlockSpec((TCB, dc), lambda i: (i, 0)),
            pl.BlockSpec((TCB, 8), lambda i: (i, 0)),
            pl.BlockSpec((TCB, 8), lambda i: (i, 0)),
        ],
        out_shape=[
            jax.ShapeDtypeStruct((n, dc), jnp.float32),
            jax.ShapeDtypeStruct((n, 8), jnp.float32),
            jax.ShapeDtypeStruct((n, 8), jnp.float32),
        ],
    )(x, W, Ss, Sd)


def _tc_mid_body(p_ref, b_ref, w_ref, ss_ref, sd_ref, h_ref, as_ref, ad_ref):
    hp = p_ref[...] + b_ref[...]
    hp = jnp.where(hp > 0, hp, jnp.exp(hp) - 1.0)
    h = jnp.dot(hp, w_ref[...], preferred_element_type=jnp.float32)
    h_ref[...] = jnp.concatenate(
        [h, jnp.zeros((h.shape[0], 128 - h.shape[1]), jnp.float32)], axis=1)
    as_ref[...] = jnp.dot(h, ss_ref[...], preferred_element_type=jnp.float32)
    ad_ref[...] = jnp.dot(h, sd_ref[...], preferred_element_type=jnp.float32)


def _tc_mid(p, b1, W2, Ss, Sd):
    n, d = p.shape
    dc = W2.shape[1]
    return pl.pallas_call(
        _tc_mid_body,
        grid=(n // TCB,),
        in_specs=[
            pl.BlockSpec((TCB, d), lambda i: (i, 0)),
            pl.BlockSpec((1, d), lambda i: (0, 0)),
            pl.BlockSpec((d, dc), lambda i: (0, 0)),
            pl.BlockSpec((dc, 8), lambda i: (0, 0)),
            pl.BlockSpec((dc, 8), lambda i: (0, 0)),
        ],
        out_specs=[
            pl.BlockSpec((TCB, 128), lambda i: (i, 0)),
            pl.BlockSpec((TCB, 8), lambda i: (i, 0)),
            pl.BlockSpec((TCB, 8), lambda i: (i, 0)),
        ],
        out_shape=[
            jax.ShapeDtypeStruct((n, 128), jnp.float32),
            jax.ShapeDtypeStruct((n, 8), jnp.float32),
            jax.ShapeDtypeStruct((n, 8), jnp.float32),
        ],
    )(p, b1, W2, Ss, Sd)


def _tc_final_body(p_ref, b_ref, o_ref):
    o_ref[...] = p_ref[...] + b_ref[...]


def _tc_final(p, b2):
    n, d = p.shape
    return pl.pallas_call(
        _tc_final_body,
        grid=(n // TCB,),
        in_specs=[
            pl.BlockSpec((TCB, d), lambda i: (i, 0)),
            pl.BlockSpec((1, d), lambda i: (0, 0)),
        ],
        out_specs=pl.BlockSpec((TCB, d), lambda i: (i, 0)),
        out_shape=jax.ShapeDtypeStruct((n, d), jnp.float32),
    )(p, b2)


def _make_sc_gat(NV, C):
    E_TILE = E_PAD // 16
    NCH = E_TILE // K
    KB = K * 8
    NT = KB // K

    mesh = plsc.VectorSubcoreMesh(core_axis_name="c", subcore_axis_name="s",
                                  num_cores=1)

    @functools.partial(
        pl.kernel,
        out_type=[
            jax.ShapeDtypeStruct((N_PAD, 128), jnp.float32),
            jax.ShapeDtypeStruct((E_PAD * 8,), jnp.float32),
        ],
        mesh=mesh,
        compiler_params=pltpu.CompilerParams(needs_layout_passes=False),
        scratch_types=[
            pltpu.VMEM_SHARED((N_PAD * 8,), jnp.float32),
            pltpu.VMEM_SHARED((N_PAD * 8,), jnp.float32),
            pltpu.VMEM_SHARED((N_PAD * 8,), jnp.float32),
            pltpu.VMEM_SHARED((N_PAD, 128), jnp.float32),
            pltpu.VMEM((1, K), jnp.int32),
            pltpu.VMEM((1, K), jnp.int32),
            pltpu.VMEM((KB,), jnp.int32),
            pltpu.VMEM((KB,), jnp.int32),
            pltpu.VMEM((NT, K), jnp.int32),
            pltpu.VMEM((KB,), jnp.float32),
            pltpu.VMEM((KB,), jnp.float32),
            pltpu.VMEM((KB,), jnp.float32),
            pltpu.VMEM((KB,), jnp.float32),
            pltpu.VMEM((K, 128), jnp.float32),
            pltpu.SemaphoreType.DMA,
            pltpu.SemaphoreType.DMA,
        ],
    )
    def sc_gat(h_hbm, as_hbm, ad_hbm, src_hbm, dst_hbm, out_hbm, ee_hbm,
               ASF, ADF, DENF, OUT, srcb, dstb, eis, eid, eid2, asr, adr,
               eef, alphab, hrows, sem1, sem2):
        s = lax.axis_index("s")
        row0 = s * ROWS_PER_TILE
        iota = lax.iota(jnp.int32, 16)
        pr = iota >> 3
        pc = iota & 7
        z0 = iota * 0
        zv = jnp.zeros((16,), jnp.float32)

        def zrow(i, _):
            @pl.when(i < KB // 16)
            def _():
                eef[pl.ds(i * 16, 16)] = zv
            for v in range(8):
                hrows[i, pl.ds(v * 16, 16)] = zv
            return 0
        lax.fori_loop(0, K, zrow, 0)

        w0 = s * W_TILE
        for o, ln in _chunks(W_TILE, KB):
            pltpu.sync_copy(as_hbm.at[pl.ds(w0 + o, ln)], asr.at[pl.ds(0, ln)])
            pltpu.sync_copy(asr.at[pl.ds(0, ln)], ASF.at[pl.ds(w0 + o, ln)])
            pltpu.sync_copy(ad_hbm.at[pl.ds(w0 + o, ln)], adr.at[pl.ds(0, ln)])
            pltpu.sync_copy(adr.at[pl.ds(0, ln)], ADF.at[pl.ds(w0 + o, ln)])
            pltpu.sync_copy(eef.at[pl.ds(0, ln)], DENF.at[pl.ds(w0 + o, ln)])
        for o, ln in _chunks(ROWS_PER_TILE, K):
            pltpu.sync_copy(hrows.at[pl.ds(0, ln)],
                            OUT.at[pl.ds(row0 + o, ln)])
        plsc.subcore_barrier()

        def p1_chunk(ch, _):
            off = s * E_TILE + ch * K
            pltpu.sync_copy(src_hbm.at[off // K], srcb)
            pltpu.sync_copy(dst_hbm.at[off // K], dstb)

            def bidx(i, _):
                k = i * 2
                sv = plsc.load_gather(srcb, [z0, pr + k])
                dv = plsc.load_gather(dstb, [z0, pr + k])
                eis[pl.ds(i * 16, 16)] = sv * 8 + pc
                ed = dv * 8 + pc
                eid[pl.ds(i * 16, 16)] = ed
                eid2[i >> 3, pl.ds((i & 7) * 16, 16)] = ed
                return 0
            lax.fori_loop(0, K // 2, bidx, 0)

            ds = []
            for j in range(NT):
                ds.append(pltpu.async_copy(
                    ASF.at[eis.at[pl.ds(j * K, K)]],
                    asr.at[pl.ds(j * K, K)], sem1))
                ds.append(pltpu.async_copy(
                    ADF.at[eid.at[pl.ds(j * K, K)]],
                    adr.at[pl.ds(j * K, K)], sem2))
            for d in ds:
                d.wait()

            def body(i, _):
                e = asr[pl.ds(i * 16, 16)] + adr[pl.ds(i * 16, 16)]
                e = jnp.maximum(e, 0.2 * e)
                eef[pl.ds(i * 16, 16)] = jnp.exp(e)
                return 0
            lax.fori_loop(0, KB // 16, body, 0)

            for j in range(NT):
                pltpu.sync_copy(eef.at[pl.ds(j * K, K)],
                                DENF.at[eid2.at[j]], add=True)
            pltpu.sync_copy(eef, ee_hbm.at[pl.ds(off * 8, KB)])
            return 0
        lax.fori_loop(0, NCH, p1_chunk, 0)
        plsc.subcore_barrier()

        def p2_chunk(ch, _):
            off = s * E_TILE + ch * K
            pltpu.sync_copy(src_hbm.at[off // K], srcb)
            pltpu.sync_copy(dst_hbm.at[off // K], dstb)

            def bidx(i, _):
                k = i * 2
                dv = plsc.load_gather(dstb, [z0, pr + k])
                eid[pl.ds(i * 16, 16)] = dv * 8 + pc
                return 0
            lax.fori_loop(0, K // 2, bidx, 0)

            d1 = pltpu.async_copy(h_hbm.at[srcb.at[0]], hrows, sem1)
            ds = []
            for j in range(NT):
                ds.append(pltpu.async_copy(
                    DENF.at[eid.at[pl.ds(j * K, K)]],
                    asr.at[pl.ds(j * K, K)], sem2))
            d1.wait()
            for d in ds:
                d.wait()
            pltpu.sync_copy(ee_hbm.at[pl.ds(off * 8, KB)], eef)

            def abody(i, _):
                ee = eef[pl.ds(i * 16, 16)]
                dn = asr[pl.ds(i * 16, 16)]
                alphab[pl.ds(i * 16, 16)] = ee / (dn + 1e-16)
                return 0
            lax.fori_loop(0, KB // 16, abody, 0)

            def mbody(k, _):
                k8 = k * 8
                for v in range(NV):
                    av = plsc.load_gather(
                        alphab,
                        [jnp.full((16,), k8 + (v * 16) // C, jnp.int32)])
                    hv = hrows[k, pl.ds(v * 16, 16)]
                    hrows[k, pl.ds(v * 16, 16)] = hv * av
                return 0
            lax.fori_loop(0, K, mbody, 0)

            pltpu.sync_copy(hrows, OUT.at[dstb.at[0]], add=True)
            return 0
        lax.fori_loop(0, NCH, p2_chunk, 0)
        plsc.subcore_barrier()

        for o, ln in _chunks(ROWS_PER_TILE, K):
            r = row0 + o
            pltpu.sync_copy(OUT.at[pl.ds(r, ln)], out_hbm.at[pl.ds(r, ln)])

    return sc_gat


_sc_gat1 = _make_sc_gat(8, C1)
_sc_gat2 = _make_sc_gat(4, C2)


def _att_mat(att, dc):
    H, _ = att.shape
    S = att[:, :, None] * jnp.eye(H, dtype=jnp.float32)[:, None, :]
    S = S.reshape(dc, H)
    return jnp.pad(S, ((0, 0), (0, 8 - H)))


def kernel(x, edge_index, W1, att_src1, att_dst1, b1, W2, att_src2, att_dst2,
           b2):
    xp = jnp.pad(x, ((0, N_PAD - N_NODES), (0, 0)))
    loops = jnp.arange(N_NODES, dtype=jnp.int32)
    fill = jnp.full((E_PAD - N_E,), N_NODES, jnp.int32)
    src = jnp.concatenate([edge_index[0].astype(jnp.int32), loops, fill])
    dst = jnp.concatenate([edge_index[1].astype(jnp.int32), loops, fill])
    src2d = src.reshape(E_PAD // K, 1, K)
    dst2d = dst.reshape(E_PAD // K, 1, K)

    Ss1 = _att_mat(att_src1, D_FEAT)
    Sd1 = _att_mat(att_dst1, D_FEAT)
    Ss2 = _att_mat(att_src2, C2)
    Sd2 = _att_mat(att_dst2, C2)

    h1, AS1, AD1 = _tc_proj(xp, W1, Ss1, Sd1)
    p1, _ = _sc_gat1(h1, AS1.reshape(-1), AD1.reshape(-1), src2d, dst2d)
    h2, AS2, AD2 = _tc_mid(p1, b1.reshape(1, -1), W2, Ss2, Sd2)
    p2, _ = _sc_gat2(h2, AS2.reshape(-1), AD2.reshape(-1), src2d, dst2d)
    b2p = jnp.pad(b2, (0, 128 - C2)).reshape(1, 128)
    out = _tc_final(p2, b2p)
    return out[:N_NODES, :C2]


# speedup vs baseline: 15.2634x; 1.1114x over previous
import functools

import jax
import jax.numpy as jnp
from jax import lax
from jax.experimental import pallas as pl
from jax.experimental.pallas import tpu as pltpu
from jax.experimental.pallas import tpu_sc as plsc

N_NODES = 10000
N_EDGES_IN = 320000
N_E = N_EDGES_IN + N_NODES
D_FEAT = 128
HEADS1 = 8
C1 = 16
C2 = 64

N_PAD = 10112
K = 128
E_PAD = 331776
ROWS_PER_TILE = N_PAD // 16
W_TILE = N_PAD * 8 // 16

TCB = 632


def _chunks(total, step):
    out = []
    o = 0
    while o < total:
        out.append((o, min(step, total - o)))
        o += step
    return out


def _tc_proj_body(x_ref, w_ref, ss_ref, sd_ref, h_ref, as_ref, ad_ref):
    h = jnp.dot(x_ref[...], w_ref[...], preferred_element_type=jnp.float32)
    h_ref[...] = h
    as_ref[...] = jnp.dot(h, ss_ref[...], preferred_element_type=jnp.float32)
    ad_ref[...] = jnp.dot(h, sd_ref[...], preferred_element_type=jnp.float32)


def _tc_proj(x, W, Ss, Sd):
    n, d = x.shape
    dc = W.shape[1]
    return pl.pallas_call(
        _tc_proj_body,
        grid=(n // TCB,),
        in_specs=[
            pl.BlockSpec((TCB, d), lambda i: (i, 0)),
            pl.BlockSpec((d, dc), lambda i: (0, 0)),
            pl.BlockSpec((dc, 8), lambda i: (0, 0)),
            pl.BlockSpec((dc, 8), lambda i: (0, 0)),
        ],
        out_specs=[
            pl.BlockSpec((TCB, dc), lambda i: (i, 0)),
            pl.BlockSpec((TCB, 8), lambda i: (i, 0)),
            pl.BlockSpec((TCB, 8), lambda i: (i, 0)),
        ],
        out_shape=[
            jax.ShapeDtypeStruct((n, dc), jnp.float32),
            jax.ShapeDtypeStruct((n, 8), jnp.float32),
            jax.ShapeDtypeStruct((n, 8), jnp.float32),
        ],
    )(x, W, Ss, Sd)


def _tc_mid_body(p_ref, b_ref, w_ref, ss_ref, sd_ref, h_ref, as_ref, ad_ref):
    hp = p_ref[...] + b_ref[...]
    hp = jnp.where(hp > 0, hp, jnp.exp(hp) - 1.0)
    h = jnp.dot(hp, w_ref[...], preferred_element_type=jnp.float32)
    h_ref[...] = jnp.concatenate(
        [h, jnp.zeros((h.shape[0], 128 - h.shape[1]), jnp.float32)], axis=1)
    as_ref[...] = jnp.dot(h, ss_ref[...], preferred_element_type=jnp.float32)
    ad_ref[...] = jnp.dot(h, sd_ref[...], preferred_element_type=jnp.float32)


def _tc_mid(p, b1, W2, Ss, Sd):
    n, d = p.shape
    dc = W2.shape[1]
    return pl.pallas_call(
        _tc_mid_body,
        grid=(n // TCB,),
        in_specs=[
            pl.BlockSpec((TCB, d), lambda i: (i, 0)),
            pl.BlockSpec((1, d), lambda i: (0, 0)),
            pl.BlockSpec((d, dc), lambda i: (0, 0)),
            pl.BlockSpec((dc, 8), lambda i: (0, 0)),
            pl.BlockSpec((dc, 8), lambda i: (0, 0)),
        ],
        out_specs=[
            pl.BlockSpec((TCB, 128), lambda i: (i, 0)),
            pl.BlockSpec((TCB, 8), lambda i: (i, 0)),
            pl.BlockSpec((TCB, 8), lambda i: (i, 0)),
        ],
        out_shape=[
            jax.ShapeDtypeStruct((n, 128), jnp.float32),
            jax.ShapeDtypeStruct((n, 8), jnp.float32),
            jax.ShapeDtypeStruct((n, 8), jnp.float32),
        ],
    )(p, b1, W2, Ss, Sd)


def _tc_final_body(p_ref, b_ref, o_ref):
    o_ref[...] = p_ref[...] + b_ref[...]


def _tc_final(p, b2):
    n, d = p.shape
    return pl.pallas_call(
        _tc_final_body,
        grid=(n // TCB,),
        in_specs=[
            pl.BlockSpec((TCB, d), lambda i: (i, 0)),
            pl.BlockSpec((1, d), lambda i: (0, 0)),
        ],
        out_specs=pl.BlockSpec((TCB, d), lambda i: (i, 0)),
        out_shape=jax.ShapeDtypeStruct((n, d), jnp.float32),
    )(p, b2)


def _make_sc_gat(NV, C, NH):
    E_TILE = E_PAD // 16
    NCH = E_TILE // K
    KB = K * NH
    NT = KB // K

    mesh = plsc.VectorSubcoreMesh(core_axis_name="c", subcore_axis_name="s",
                                  num_cores=1)

    @functools.partial(
        pl.kernel,
        out_type=[
            jax.ShapeDtypeStruct((N_PAD, 128), jnp.float32),
            jax.ShapeDtypeStruct((E_PAD * NH,), jnp.float32),
        ],
        mesh=mesh,
        compiler_params=pltpu.CompilerParams(needs_layout_passes=False),
        scratch_types=[
            pltpu.VMEM_SHARED((N_PAD * 8,), jnp.float32),
            pltpu.VMEM_SHARED((N_PAD * 8,), jnp.float32),
            pltpu.VMEM_SHARED((N_PAD * 8,), jnp.float32),
            pltpu.VMEM_SHARED((N_PAD, 128), jnp.float32),
            pltpu.VMEM((1, K), jnp.int32),
            pltpu.VMEM((1, K), jnp.int32),
            pltpu.VMEM((KB,), jnp.int32),
            pltpu.VMEM((KB,), jnp.int32),
            pltpu.VMEM((NT, K), jnp.int32),
            pltpu.VMEM((KB,), jnp.float32),
            pltpu.VMEM((KB,), jnp.float32),
            pltpu.VMEM((KB,), jnp.float32),
            pltpu.VMEM((KB,), jnp.float32),
            pltpu.VMEM((K, 128), jnp.float32),
            pltpu.SemaphoreType.DMA,
            pltpu.SemaphoreType.DMA,
        ],
    )
    def sc_gat(h_hbm, as_hbm, ad_hbm, src_hbm, dst_hbm, out_hbm, ee_hbm,
               ASF, ADF, DENF, OUT, srcb, dstb, eis, eid, eid2, asr, adr,
               eef, alphab, hrows, sem1, sem2):
        s = lax.axis_index("s")
        row0 = s * ROWS_PER_TILE
        iota = lax.iota(jnp.int32, 16)
        pr = iota >> 3
        pc = iota & 7
        z0 = iota * 0
        zv = jnp.zeros((16,), jnp.float32)

        def zrow(i, _):
            @pl.when(i < KB // 16)
            def _():
                eef[pl.ds(i * 16, 16)] = zv
            for v in range(8):
                hrows[i, pl.ds(v * 16, 16)] = zv
            return 0
        lax.fori_loop(0, K, zrow, 0)

        w0 = s * W_TILE
        for o, ln in _chunks(W_TILE, KB):
            pltpu.sync_copy(as_hbm.at[pl.ds(w0 + o, ln)], asr.at[pl.ds(0, ln)])
            pltpu.sync_copy(asr.at[pl.ds(0, ln)], ASF.at[pl.ds(w0 + o, ln)])
            pltpu.sync_copy(ad_hbm.at[pl.ds(w0 + o, ln)], adr.at[pl.ds(0, ln)])
            pltpu.sync_copy(adr.at[pl.ds(0, ln)], ADF.at[pl.ds(w0 + o, ln)])
            pltpu.sync_copy(eef.at[pl.ds(0, ln)], DENF.at[pl.ds(w0 + o, ln)])
        for o, ln in _chunks(ROWS_PER_TILE, K):
            pltpu.sync_copy(hrows.at[pl.ds(0, ln)],
                            OUT.at[pl.ds(row0 + o, ln)])
        plsc.subcore_barrier()

        def p1_chunk(ch, _):
            off = s * E_TILE + ch * K
            pltpu.sync_copy(src_hbm.at[off // K], srcb)
            pltpu.sync_copy(dst_hbm.at[off // K], dstb)

            if NH == 8:
                def bidx(i, _):
                    k = i * 2
                    sv = plsc.load_gather(srcb, [z0, pr + k])
                    dv = plsc.load_gather(dstb, [z0, pr + k])
                    eis[pl.ds(i * 16, 16)] = sv * 8 + pc
                    ed = dv * 8 + pc
                    eid[pl.ds(i * 16, 16)] = ed
                    eid2[i >> 3, pl.ds((i & 7) * 16, 16)] = ed
                    return 0
                nb = K // 2
            else:
                def bidx(i, _):
                    sv = plsc.load_gather(srcb, [z0, iota + i * 16])
                    dv = plsc.load_gather(dstb, [z0, iota + i * 16])
                    eis[pl.ds(i * 16, 16)] = sv * 8
                    ed = dv * 8
                    eid[pl.ds(i * 16, 16)] = ed
                    eid2[0, pl.ds(i * 16, 16)] = ed
                    return 0
                nb = K // 16
            lax.fori_loop(0, nb, bidx, 0)

            ds = []
            for j in range(NT):
                ds.append(pltpu.async_copy(
                    ASF.at[eis.at[pl.ds(j * K, K)]],
                    asr.at[pl.ds(j * K, K)], sem1))
                ds.append(pltpu.async_copy(
                    ADF.at[eid.at[pl.ds(j * K, K)]],
                    adr.at[pl.ds(j * K, K)], sem2))
            for d in ds:
                d.wait()

            def body(i, _):
                e = asr[pl.ds(i * 16, 16)] + adr[pl.ds(i * 16, 16)]
                e = jnp.maximum(e, 0.2 * e)
                eef[pl.ds(i * 16, 16)] = jnp.exp(e)
                return 0
            lax.fori_loop(0, KB // 16, body, 0)

            for j in range(NT):
                pltpu.sync_copy(eef.at[pl.ds(j * K, K)],
                                DENF.at[eid2.at[j]], add=True)
            pltpu.sync_copy(eef, ee_hbm.at[pl.ds(off * NH, KB)])
            return 0
        lax.fori_loop(0, NCH, p1_chunk, 0)
        plsc.subcore_barrier()

        def p2_chunk(ch, _):
            off = s * E_TILE + ch * K
            pltpu.sync_copy(src_hbm.at[off // K], srcb)
            pltpu.sync_copy(dst_hbm.at[off // K], dstb)

            if NH == 8:
                def bidx(i, _):
                    k = i * 2
                    dv = plsc.load_gather(dstb, [z0, pr + k])
                    eid[pl.ds(i * 16, 16)] = dv * 8 + pc
                    return 0
                nb = K // 2
            else:
                def bidx(i, _):
                    dv = plsc.load_gather(dstb, [z0, iota + i * 16])
                    eid[pl.ds(i * 16, 16)] = dv * 8
                    return 0
                nb = K // 16
            lax.fori_loop(0, nb, bidx, 0)

            d1 = pltpu.async_copy(h_hbm.at[srcb.at[0]], hrows, sem1)
            ds = []
            for j in range(NT):
                ds.append(pltpu.async_copy(
                    DENF.at[eid.at[pl.ds(j * K, K)]],
                    asr.at[pl.ds(j * K, K)], sem2))
            d1.wait()
            for d in ds:
                d.wait()
            pltpu.sync_copy(ee_hbm.at[pl.ds(off * NH, KB)], eef)

            def abody(i, _):
                ee = eef[pl.ds(i * 16, 16)]
                dn = asr[pl.ds(i * 16, 16)]
                alphab[pl.ds(i * 16, 16)] = ee / (dn + 1e-16)
                return 0
            lax.fori_loop(0, KB // 16, abody, 0)

            def mbody(k, _):
                k8 = k * NH
                for v in range(NV):
                    av = plsc.load_gather(
                        alphab,
                        [jnp.full((16,), k8 + (v * 16) // C, jnp.int32)])
                    hv = hrows[k, pl.ds(v * 16, 16)]
                    hrows[k, pl.ds(v * 16, 16)] = hv * av
                return 0
            lax.fori_loop(0, K, mbody, 0)

            pltpu.sync_copy(hrows, OUT.at[dstb.at[0]], add=True)
            return 0
        lax.fori_loop(0, NCH, p2_chunk, 0)
        plsc.subcore_barrier()

        for o, ln in _chunks(ROWS_PER_TILE, K):
            r = row0 + o
            pltpu.sync_copy(OUT.at[pl.ds(r, ln)], out_hbm.at[pl.ds(r, ln)])

    return sc_gat


_sc_gat1 = _make_sc_gat(8, C1, 8)
_sc_gat2 = _make_sc_gat(4, C2, 1)


def _att_mat(att, dc):
    H, _ = att.shape
    S = att[:, :, None] * jnp.eye(H, dtype=jnp.float32)[:, None, :]
    S = S.reshape(dc, H)
    return jnp.pad(S, ((0, 0), (0, 8 - H)))


def kernel(x, edge_index, W1, att_src1, att_dst1, b1, W2, att_src2, att_dst2,
           b2):
    xp = jnp.pad(x, ((0, N_PAD - N_NODES), (0, 0)))
    loops = jnp.arange(N_NODES, dtype=jnp.int32)
    fill = jnp.full((E_PAD - N_E,), N_NODES, jnp.int32)
    src = jnp.concatenate([edge_index[0].astype(jnp.int32), loops, fill])
    dst = jnp.concatenate([edge_index[1].astype(jnp.int32), loops, fill])
    src2d = src.reshape(E_PAD // K, 1, K)
    dst2d = dst.reshape(E_PAD // K, 1, K)

    Ss1 = _att_mat(att_src1, D_FEAT)
    Sd1 = _att_mat(att_dst1, D_FEAT)
    Ss2 = _att_mat(att_src2, C2)
    Sd2 = _att_mat(att_dst2, C2)

    h1, AS1, AD1 = _tc_proj(xp, W1, Ss1, Sd1)
    p1, _ = _sc_gat1(h1, AS1.reshape(-1), AD1.reshape(-1), src2d, dst2d)
    h2, AS2, AD2 = _tc_mid(p1, b1.reshape(1, -1), W2, Ss2, Sd2)
    p2, _ = _sc_gat2(h2, AS2.reshape(-1), AD2.reshape(-1), src2d, dst2d)
    b2p = jnp.pad(b2, (0, 128 - C2)).reshape(1, 128)
    out = _tc_final(p2, b2p)
    return out[:N_NODES, :C2]


# speedup vs baseline: 23.3904x; 1.5324x over previous
import functools

import jax
import jax.numpy as jnp
from jax import lax
from jax.experimental import pallas as pl
from jax.experimental.pallas import tpu as pltpu
from jax.experimental.pallas import tpu_sc as plsc

N_NODES = 10000
N_EDGES_IN = 320000
N_E = N_EDGES_IN + N_NODES
D_FEAT = 128
HEADS1 = 8
C1 = 16
C2 = 64

N_PAD = 10112
K = 128
E_PAD = 331776
ROWS_PER_TILE = N_PAD // 16
W_TILE = N_PAD * 8 // 16

TCB = 632


def _chunks(total, step):
    out = []
    o = 0
    while o < total:
        out.append((o, min(step, total - o)))
        o += step
    return out


def _tc_proj_body(x_ref, w_ref, ss_ref, sd_ref, h_ref, as_ref, ad_ref):
    h = jnp.dot(x_ref[...], w_ref[...], preferred_element_type=jnp.float32)
    h_ref[...] = h
    as_ref[...] = jnp.dot(h, ss_ref[...], preferred_element_type=jnp.float32)
    ad_ref[...] = jnp.dot(h, sd_ref[...], preferred_element_type=jnp.float32)


def _tc_proj(x, W, Ss, Sd):
    n, d = x.shape
    dc = W.shape[1]
    return pl.pallas_call(
        _tc_proj_body,
        grid=(n // TCB,),
        in_specs=[
            pl.BlockSpec((TCB, d), lambda i: (i, 0)),
            pl.BlockSpec((d, dc), lambda i: (0, 0)),
            pl.BlockSpec((dc, 8), lambda i: (0, 0)),
            pl.BlockSpec((dc, 8), lambda i: (0, 0)),
        ],
        out_specs=[
            pl.BlockSpec((TCB, dc), lambda i: (i, 0)),
            pl.BlockSpec((TCB, 8), lambda i: (i, 0)),
            pl.BlockSpec((TCB, 8), lambda i: (i, 0)),
        ],
        out_shape=[
            jax.ShapeDtypeStruct((n, dc), jnp.float32),
            jax.ShapeDtypeStruct((n, 8), jnp.float32),
            jax.ShapeDtypeStruct((n, 8), jnp.float32),
        ],
    )(x, W, Ss, Sd)


def _tc_mid_body(p_ref, b_ref, w_ref, ss_ref, sd_ref, h_ref, as_ref, ad_ref):
    hp = p_ref[...] + b_ref[...]
    hp = jnp.where(hp > 0, hp, jnp.exp(hp) - 1.0)
    h = jnp.dot(hp, w_ref[...], preferred_element_type=jnp.float32)
    h_ref[...] = jnp.concatenate(
        [h, jnp.zeros((h.shape[0], 128 - h.shape[1]), jnp.float32)], axis=1)
    as_ref[...] = jnp.dot(h, ss_ref[...], preferred_element_type=jnp.float32)
    ad_ref[...] = jnp.dot(h, sd_ref[...], preferred_element_type=jnp.float32)


def _tc_mid(p, b1, W2, Ss, Sd):
    n, d = p.shape
    dc = W2.shape[1]
    return pl.pallas_call(
        _tc_mid_body,
        grid=(n // TCB,),
        in_specs=[
            pl.BlockSpec((TCB, d), lambda i: (i, 0)),
            pl.BlockSpec((1, d), lambda i: (0, 0)),
            pl.BlockSpec((d, dc), lambda i: (0, 0)),
            pl.BlockSpec((dc, 8), lambda i: (0, 0)),
            pl.BlockSpec((dc, 8), lambda i: (0, 0)),
        ],
        out_specs=[
            pl.BlockSpec((TCB, 128), lambda i: (i, 0)),
            pl.BlockSpec((TCB, 8), lambda i: (i, 0)),
            pl.BlockSpec((TCB, 8), lambda i: (i, 0)),
        ],
        out_shape=[
            jax.ShapeDtypeStruct((n, 128), jnp.float32),
            jax.ShapeDtypeStruct((n, 8), jnp.float32),
            jax.ShapeDtypeStruct((n, 8), jnp.float32),
        ],
    )(p, b1, W2, Ss, Sd)


def _tc_final_body(p_ref, b_ref, o_ref):
    o_ref[...] = p_ref[...] + b_ref[...]


def _tc_final(p, b2):
    n, d = p.shape
    return pl.pallas_call(
        _tc_final_body,
        grid=(n // TCB,),
        in_specs=[
            pl.BlockSpec((TCB, d), lambda i: (i, 0)),
            pl.BlockSpec((1, d), lambda i: (0, 0)),
        ],
        out_specs=pl.BlockSpec((TCB, d), lambda i: (i, 0)),
        out_shape=jax.ShapeDtypeStruct((n, d), jnp.float32),
    )(p, b2)


def _make_sc_gat(NV, C, NH):
    E_TILE = E_PAD // 16
    NCH = E_TILE // K
    KB = K * NH
    NT = KB // K

    mesh = plsc.VectorSubcoreMesh(core_axis_name="c", subcore_axis_name="s",
                                  num_cores=1)

    @functools.partial(
        pl.kernel,
        out_type=[
            jax.ShapeDtypeStruct((N_PAD, 128), jnp.float32),
            jax.ShapeDtypeStruct((E_PAD * NH,), jnp.float32),
        ],
        mesh=mesh,
        compiler_params=pltpu.CompilerParams(needs_layout_passes=False),
        scratch_types=[
            pltpu.VMEM_SHARED((N_PAD * 8,), jnp.float32),
            pltpu.VMEM_SHARED((N_PAD * 8,), jnp.float32),
            pltpu.VMEM_SHARED((N_PAD * 8,), jnp.float32),
            pltpu.VMEM_SHARED((N_PAD, 128), jnp.float32),
            pltpu.VMEM((1, K), jnp.int32),
            pltpu.VMEM((1, K), jnp.int32),
            pltpu.VMEM((KB,), jnp.int32),
            pltpu.VMEM((KB,), jnp.int32),
            pltpu.VMEM((NT, K), jnp.int32),
            pltpu.VMEM((KB,), jnp.float32),
            pltpu.VMEM((KB,), jnp.float32),
            pltpu.VMEM((KB,), jnp.float32),
            pltpu.VMEM((KB,), jnp.float32),
            pltpu.VMEM((K, 128), jnp.float32),
            pltpu.SemaphoreType.DMA,
            pltpu.SemaphoreType.DMA,
        ],
    )
    def sc_gat(h_hbm, as_hbm, ad_hbm, src_hbm, dst_hbm, out_hbm, ee_hbm,
               ASF, ADF, DENF, OUT, srcb, dstb, eis, eid, eid2, asr, adr,
               eef, alphab, hrows, sem1, sem2):
        s = lax.axis_index("s")
        row0 = s * ROWS_PER_TILE
        iota = lax.iota(jnp.int32, 16)
        pr = iota >> 3
        pc = iota & 7
        z0 = iota * 0
        zv = jnp.zeros((16,), jnp.float32)

        def zrow(i, _):
            @pl.when(i < KB // 16)
            def _():
                eef[pl.ds(i * 16, 16)] = zv
            for v in range(8):
                hrows[i, pl.ds(v * 16, 16)] = zv
            return 0
        lax.fori_loop(0, K, zrow, 0)

        w0 = s * W_TILE
        for o, ln in _chunks(W_TILE, KB):
            pltpu.sync_copy(as_hbm.at[pl.ds(w0 + o, ln)], asr.at[pl.ds(0, ln)])
            pltpu.sync_copy(asr.at[pl.ds(0, ln)], ASF.at[pl.ds(w0 + o, ln)])
            pltpu.sync_copy(ad_hbm.at[pl.ds(w0 + o, ln)], adr.at[pl.ds(0, ln)])
            pltpu.sync_copy(adr.at[pl.ds(0, ln)], ADF.at[pl.ds(w0 + o, ln)])
            pltpu.sync_copy(eef.at[pl.ds(0, ln)], DENF.at[pl.ds(w0 + o, ln)])
        for o, ln in _chunks(ROWS_PER_TILE, K):
            pltpu.sync_copy(hrows.at[pl.ds(0, ln)],
                            OUT.at[pl.ds(row0 + o, ln)])
        plsc.subcore_barrier()

        def p1_chunk(ch, _):
            off = s * E_TILE + ch * K
            pltpu.sync_copy(src_hbm.at[off // K], srcb)
            pltpu.sync_copy(dst_hbm.at[off // K], dstb)

            if NH == 8:
                def bidx(i):
                    k = i * 2
                    sv = plsc.load_gather(srcb, [z0, pr + k])
                    dv = plsc.load_gather(dstb, [z0, pr + k])
                    eis[pl.ds(i * 16, 16)] = sv * 8 + pc
                    ed = dv * 8 + pc
                    eid[pl.ds(i * 16, 16)] = ed
                    eid2[i >> 3, pl.ds((i & 7) * 16, 16)] = ed
                nb = K // 2
            else:
                def bidx(i):
                    sv = plsc.load_gather(srcb, [z0, iota + i * 16])
                    dv = plsc.load_gather(dstb, [z0, iota + i * 16])
                    eis[pl.ds(i * 16, 16)] = sv * 8
                    ed = dv * 8
                    eid[pl.ds(i * 16, 16)] = ed
                    eid2[0, pl.ds(i * 16, 16)] = ed
                nb = K // 16
            plsc.parallel_loop(0, nb, unroll=4)(bidx)

            ds = []
            for j in range(NT):
                ds.append(pltpu.async_copy(
                    ASF.at[eis.at[pl.ds(j * K, K)]],
                    asr.at[pl.ds(j * K, K)], sem1))
                ds.append(pltpu.async_copy(
                    ADF.at[eid.at[pl.ds(j * K, K)]],
                    adr.at[pl.ds(j * K, K)], sem2))
            for d in ds:
                d.wait()

            def body(i):
                e = asr[pl.ds(i * 16, 16)] + adr[pl.ds(i * 16, 16)]
                e = jnp.maximum(e, 0.2 * e)
                eef[pl.ds(i * 16, 16)] = jnp.exp(e)
            plsc.parallel_loop(0, KB // 16, unroll=4)(body)

            ds2 = []
            for j in range(NT):
                ds2.append(pltpu.async_copy(
                    eef.at[pl.ds(j * K, K)], DENF.at[eid2.at[j]], sem2,
                    add=True))
            pltpu.sync_copy(eef, ee_hbm.at[pl.ds(off * NH, KB)])
            for d in ds2:
                d.wait()
            return 0
        lax.fori_loop(0, NCH, p1_chunk, 0)
        plsc.subcore_barrier()

        def p2_chunk(ch, _):
            off = s * E_TILE + ch * K
            pltpu.sync_copy(src_hbm.at[off // K], srcb)
            pltpu.sync_copy(dst_hbm.at[off // K], dstb)

            if NH == 8:
                def bidx(i):
                    k = i * 2
                    dv = plsc.load_gather(dstb, [z0, pr + k])
                    eid[pl.ds(i * 16, 16)] = dv * 8 + pc
                nb = K // 2
            else:
                def bidx(i):
                    dv = plsc.load_gather(dstb, [z0, iota + i * 16])
                    eid[pl.ds(i * 16, 16)] = dv * 8
                nb = K // 16
            plsc.parallel_loop(0, nb, unroll=4)(bidx)

            d1 = pltpu.async_copy(h_hbm.at[srcb.at[0]], hrows, sem1)
            ds = []
            for j in range(NT):
                ds.append(pltpu.async_copy(
                    DENF.at[eid.at[pl.ds(j * K, K)]],
                    asr.at[pl.ds(j * K, K)], sem2))
            d1.wait()
            for d in ds:
                d.wait()
            pltpu.sync_copy(ee_hbm.at[pl.ds(off * NH, KB)], eef)

            def abody(i):
                ee = eef[pl.ds(i * 16, 16)]
                dn = asr[pl.ds(i * 16, 16)]
                alphab[pl.ds(i * 16, 16)] = ee / (dn + 1e-16)
            plsc.parallel_loop(0, KB // 16, unroll=4)(abody)

            def mbody(k):
                k8 = k * NH
                for v in range(NV):
                    av = plsc.load_gather(
                        alphab,
                        [jnp.full((16,), k8 + (v * 16) // C, jnp.int32)])
                    hv = hrows[k, pl.ds(v * 16, 16)]
                    hrows[k, pl.ds(v * 16, 16)] = hv * av
            plsc.parallel_loop(0, K, unroll=2)(mbody)

            pltpu.sync_copy(hrows, OUT.at[dstb.at[0]], add=True)
            return 0
        lax.fori_loop(0, NCH, p2_chunk, 0)
        plsc.subcore_barrier()

        for o, ln in _chunks(ROWS_PER_TILE, K):
            r = row0 + o
            pltpu.sync_copy(OUT.at[pl.ds(r, ln)], out_hbm.at[pl.ds(r, ln)])

    return sc_gat


_sc_gat1 = _make_sc_gat(8, C1, 8)
_sc_gat2 = _make_sc_gat(4, C2, 1)


def _att_mat(att, dc):
    H, _ = att.shape
    S = att[:, :, None] * jnp.eye(H, dtype=jnp.float32)[:, None, :]
    S = S.reshape(dc, H)
    return jnp.pad(S, ((0, 0), (0, 8 - H)))


def kernel(x, edge_index, W1, att_src1, att_dst1, b1, W2, att_src2, att_dst2,
           b2):
    xp = jnp.pad(x, ((0, N_PAD - N_NODES), (0, 0)))
    loops = jnp.arange(N_NODES, dtype=jnp.int32)
    fill = jnp.full((E_PAD - N_E,), N_NODES, jnp.int32)
    src = jnp.concatenate([edge_index[0].astype(jnp.int32), loops, fill])
    dst = jnp.concatenate([edge_index[1].astype(jnp.int32), loops, fill])
    src2d = src.reshape(E_PAD // K, 1, K)
    dst2d = dst.reshape(E_PAD // K, 1, K)

    Ss1 = _att_mat(att_src1, D_FEAT)
    Sd1 = _att_mat(att_dst1, D_FEAT)
    Ss2 = _att_mat(att_src2, C2)
    Sd2 = _att_mat(att_dst2, C2)

    h1, AS1, AD1 = _tc_proj(xp, W1, Ss1, Sd1)
    p1, _ = _sc_gat1(h1, AS1.reshape(-1), AD1.reshape(-1), src2d, dst2d)
    h2, AS2, AD2 = _tc_mid(p1, b1.reshape(1, -1), W2, Ss2, Sd2)
    p2, _ = _sc_gat2(h2, AS2.reshape(-1), AD2.reshape(-1), src2d, dst2d)
    b2p = jnp.pad(b2, (0, 128 - C2)).reshape(1, 128)
    out = _tc_final(p2, b2p)
    return out[:N_NODES, :C2]


# speedup vs baseline: 25.8963x; 1.1071x over previous
import functools

import jax
import jax.numpy as jnp
from jax import lax
from jax.experimental import pallas as pl
from jax.experimental.pallas import tpu as pltpu
from jax.experimental.pallas import tpu_sc as plsc

N_NODES = 10000
N_EDGES_IN = 320000
N_E = N_EDGES_IN + N_NODES
D_FEAT = 128
HEADS1 = 8
C1 = 16
C2 = 64

N_PAD = 10112
K = 128
E_PAD = 331776
ROWS_PER_TILE = N_PAD // 16
W_TILE = N_PAD * 8 // 16

TCB = 632


def _chunks(total, step):
    out = []
    o = 0
    while o < total:
        out.append((o, min(step, total - o)))
        o += step
    return out


def _tc_proj_body(x_ref, w_ref, ss_ref, sd_ref, h_ref, as_ref, ad_ref):
    h = jnp.dot(x_ref[...], w_ref[...], preferred_element_type=jnp.float32)
    h_ref[...] = h
    as_ref[...] = jnp.dot(h, ss_ref[...], preferred_element_type=jnp.float32)
    ad_ref[...] = jnp.dot(h, sd_ref[...], preferred_element_type=jnp.float32)


def _tc_proj(x, W, Ss, Sd):
    n, d = x.shape
    dc = W.shape[1]
    return pl.pallas_call(
        _tc_proj_body,
        grid=(n // TCB,),
        in_specs=[
            pl.BlockSpec((TCB, d), lambda i: (i, 0)),
            pl.BlockSpec((d, dc), lambda i: (0, 0)),
            pl.BlockSpec((dc, 8), lambda i: (0, 0)),
            pl.BlockSpec((dc, 8), lambda i: (0, 0)),
        ],
        out_specs=[
            pl.BlockSpec((TCB, dc), lambda i: (i, 0)),
            pl.BlockSpec((TCB, 8), lambda i: (i, 0)),
            pl.BlockSpec((TCB, 8), lambda i: (i, 0)),
        ],
        out_shape=[
            jax.ShapeDtypeStruct((n, dc), jnp.float32),
            jax.ShapeDtypeStruct((n, 8), jnp.float32),
            jax.ShapeDtypeStruct((n, 8), jnp.float32),
        ],
    )(x, W, Ss, Sd)


def _tc_mid_body(p_ref, b_ref, w_ref, ss_ref, sd_ref, h_ref, as_ref, ad_ref):
    hp = p_ref[...] + b_ref[...]
    hp = jnp.where(hp > 0, hp, jnp.exp(hp) - 1.0)
    h = jnp.dot(hp, w_ref[...], preferred_element_type=jnp.float32)
    h_ref[...] = jnp.concatenate(
        [h, jnp.zeros((h.shape[0], 128 - h.shape[1]), jnp.float32)], axis=1)
    as_ref[...] = jnp.dot(h, ss_ref[...], preferred_element_type=jnp.float32)
    ad_ref[...] = jnp.dot(h, sd_ref[...], preferred_element_type=jnp.float32)


def _tc_mid(p, b1, W2, Ss, Sd):
    n, d = p.shape
    dc = W2.shape[1]
    return pl.pallas_call(
        _tc_mid_body,
        grid=(n // TCB,),
        in_specs=[
            pl.BlockSpec((TCB, d), lambda i: (i, 0)),
            pl.BlockSpec((1, d), lambda i: (0, 0)),
            pl.BlockSpec((d, dc), lambda i: (0, 0)),
            pl.BlockSpec((dc, 8), lambda i: (0, 0)),
            pl.BlockSpec((dc, 8), lambda i: (0, 0)),
        ],
        out_specs=[
            pl.BlockSpec((TCB, 128), lambda i: (i, 0)),
            pl.BlockSpec((TCB, 8), lambda i: (i, 0)),
            pl.BlockSpec((TCB, 8), lambda i: (i, 0)),
        ],
        out_shape=[
            jax.ShapeDtypeStruct((n, 128), jnp.float32),
            jax.ShapeDtypeStruct((n, 8), jnp.float32),
            jax.ShapeDtypeStruct((n, 8), jnp.float32),
        ],
    )(p, b1, W2, Ss, Sd)


def _tc_final_body(p_ref, b_ref, o_ref):
    o_ref[...] = p_ref[...] + b_ref[...]


def _tc_final(p, b2):
    n, d = p.shape
    return pl.pallas_call(
        _tc_final_body,
        grid=(n // TCB,),
        in_specs=[
            pl.BlockSpec((TCB, d), lambda i: (i, 0)),
            pl.BlockSpec((1, d), lambda i: (0, 0)),
        ],
        out_specs=pl.BlockSpec((TCB, d), lambda i: (i, 0)),
        out_shape=jax.ShapeDtypeStruct((n, d), jnp.float32),
    )(p, b2)


def _make_sc_gat(NV, C, NH):
    E_TILE = E_PAD // 16
    NCH = E_TILE // K
    KB = K * NH
    NT = KB // K

    mesh = plsc.VectorSubcoreMesh(core_axis_name="c", subcore_axis_name="s",
                                  num_cores=1)

    @functools.partial(
        pl.kernel,
        out_type=[
            jax.ShapeDtypeStruct((N_PAD, 128), jnp.float32),
            jax.ShapeDtypeStruct((E_PAD * NH,), jnp.float32),
        ],
        mesh=mesh,
        compiler_params=pltpu.CompilerParams(needs_layout_passes=False),
        scratch_types=[
            pltpu.VMEM_SHARED((N_PAD * 8,), jnp.float32),
            pltpu.VMEM_SHARED((N_PAD * 8,), jnp.float32),
            pltpu.VMEM_SHARED((N_PAD * 8,), jnp.float32),
            pltpu.VMEM_SHARED((N_PAD, 128), jnp.float32),
            pltpu.VMEM((2, K // 2), jnp.int32),
            pltpu.VMEM((2, K // 2), jnp.int32),
            pltpu.VMEM((KB,), jnp.int32),
            pltpu.VMEM((KB,), jnp.int32),
            pltpu.VMEM((NT, K), jnp.int32),
            pltpu.VMEM((KB,), jnp.float32),
            pltpu.VMEM((KB,), jnp.float32),
            pltpu.VMEM((KB,), jnp.float32),
            pltpu.VMEM((KB,), jnp.float32),
            pltpu.VMEM((K // 2, 128), jnp.float32),
            pltpu.VMEM((K // 2, 128), jnp.float32),
            pltpu.SemaphoreType.DMA,
            pltpu.SemaphoreType.DMA,
            pltpu.SemaphoreType.DMA,
            pltpu.SemaphoreType.DMA,
        ],
    )
    def sc_gat(h_hbm, as_hbm, ad_hbm, src_hbm, dst_hbm, out_hbm, ee_hbm,
               ASF, ADF, DENF, OUT, srcb, dstb, eis, eid, eid2, asr, adr,
               eef, alphab, hrowsA, hrowsB, sem1, sem2, sem3, sem4):
        s = lax.axis_index("s")
        row0 = s * ROWS_PER_TILE
        iota = lax.iota(jnp.int32, 16)
        pr = iota >> 3
        pc = iota & 7
        z0 = iota * 0
        zv = jnp.zeros((16,), jnp.float32)

        def zrow(i, _):
            @pl.when(i < KB // 16)
            def _():
                eef[pl.ds(i * 16, 16)] = zv
            @pl.when(i < K // 2)
            def _():
                for v in range(8):
                    hrowsA[i, pl.ds(v * 16, 16)] = zv
                    hrowsB[i, pl.ds(v * 16, 16)] = zv
            return 0
        lax.fori_loop(0, K, zrow, 0)

        w0 = s * W_TILE
        for o, ln in _chunks(W_TILE, KB):
            pltpu.sync_copy(as_hbm.at[pl.ds(w0 + o, ln)], asr.at[pl.ds(0, ln)])
            pltpu.sync_copy(asr.at[pl.ds(0, ln)], ASF.at[pl.ds(w0 + o, ln)])
            pltpu.sync_copy(ad_hbm.at[pl.ds(w0 + o, ln)], adr.at[pl.ds(0, ln)])
            pltpu.sync_copy(adr.at[pl.ds(0, ln)], ADF.at[pl.ds(w0 + o, ln)])
            pltpu.sync_copy(eef.at[pl.ds(0, ln)], DENF.at[pl.ds(w0 + o, ln)])
        for o, ln in _chunks(ROWS_PER_TILE, K // 2):
            pltpu.sync_copy(hrowsA.at[pl.ds(0, ln)],
                            OUT.at[pl.ds(row0 + o, ln)])
        plsc.subcore_barrier()

        def p1_chunk(ch, _):
            off = s * E_TILE + ch * K
            pltpu.sync_copy(src_hbm.at[off // K], srcb)
            pltpu.sync_copy(dst_hbm.at[off // K], dstb)

            if NH == 8:
                def bidx(i):
                    k = i * 2
                    lane = pr + k
                    sv = plsc.load_gather(srcb, [lane >> 6, lane & 63])
                    dv = plsc.load_gather(dstb, [lane >> 6, lane & 63])
                    eis[pl.ds(i * 16, 16)] = sv * 8 + pc
                    ed = dv * 8 + pc
                    eid[pl.ds(i * 16, 16)] = ed
                    eid2[i >> 3, pl.ds((i & 7) * 16, 16)] = ed
                nb = K // 2
            else:
                def bidx(i):
                    lane = iota + i * 16
                    sv = plsc.load_gather(srcb, [lane >> 6, lane & 63])
                    dv = plsc.load_gather(dstb, [lane >> 6, lane & 63])
                    eis[pl.ds(i * 16, 16)] = sv * 8
                    ed = dv * 8
                    eid[pl.ds(i * 16, 16)] = ed
                    eid2[0, pl.ds(i * 16, 16)] = ed
                nb = K // 16
            plsc.parallel_loop(0, nb, unroll=4)(bidx)

            ds = []
            for j in range(NT):
                ds.append(pltpu.async_copy(
                    ASF.at[eis.at[pl.ds(j * K, K)]],
                    asr.at[pl.ds(j * K, K)], sem1))
                ds.append(pltpu.async_copy(
                    ADF.at[eid.at[pl.ds(j * K, K)]],
                    adr.at[pl.ds(j * K, K)], sem2))
            for d in ds:
                d.wait()

            def body(i):
                e = asr[pl.ds(i * 16, 16)] + adr[pl.ds(i * 16, 16)]
                e = jnp.maximum(e, 0.2 * e)
                eef[pl.ds(i * 16, 16)] = jnp.exp(e)
            plsc.parallel_loop(0, KB // 16, unroll=4)(body)

            ds2 = []
            for j in range(NT):
                ds2.append(pltpu.async_copy(
                    eef.at[pl.ds(j * K, K)], DENF.at[eid2.at[j]], sem2,
                    add=True))
            pltpu.sync_copy(eef, ee_hbm.at[pl.ds(off * NH, KB)])
            for d in ds2:
                d.wait()
            return 0
        lax.fori_loop(0, NCH, p1_chunk, 0)
        plsc.subcore_barrier()

        def p2_chunk(ch, _):
            off = s * E_TILE + ch * K
            pltpu.sync_copy(src_hbm.at[off // K], srcb)
            pltpu.sync_copy(dst_hbm.at[off // K], dstb)

            if NH == 8:
                def bidx(i):
                    lane = pr + i * 2
                    dv = plsc.load_gather(dstb, [lane >> 6, lane & 63])
                    eid[pl.ds(i * 16, 16)] = dv * 8 + pc
                nb = K // 2
            else:
                def bidx(i):
                    lane = iota + i * 16
                    dv = plsc.load_gather(dstb, [lane >> 6, lane & 63])
                    eid[pl.ds(i * 16, 16)] = dv * 8
                nb = K // 16
            plsc.parallel_loop(0, nb, unroll=4)(bidx)

            dA = pltpu.async_copy(h_hbm.at[srcb.at[0]], hrowsA, sem1)
            dB = pltpu.async_copy(h_hbm.at[srcb.at[1]], hrowsB, sem3)
            ds = []
            for j in range(NT):
                ds.append(pltpu.async_copy(
                    DENF.at[eid.at[pl.ds(j * K, K)]],
                    asr.at[pl.ds(j * K, K)], sem2))
            for d in ds:
                d.wait()
            pltpu.sync_copy(ee_hbm.at[pl.ds(off * NH, KB)], eef)

            def abody(i):
                ee = eef[pl.ds(i * 16, 16)]
                dn = asr[pl.ds(i * 16, 16)]
                alphab[pl.ds(i * 16, 16)] = ee / (dn + 1e-16)
            plsc.parallel_loop(0, KB // 16, unroll=4)(abody)

            def mk_mbody(buf, kofs):
                def mbody(k):
                    k8 = (k + kofs) * NH
                    for v in range(NV):
                        av = plsc.load_gather(
                            alphab,
                            [jnp.full((16,), k8 + (v * 16) // C, jnp.int32)])
                        hv = buf[k, pl.ds(v * 16, 16)]
                        buf[k, pl.ds(v * 16, 16)] = hv * av
                return mbody

            dA.wait()
            plsc.parallel_loop(0, K // 2, unroll=2)(mk_mbody(hrowsA, 0))
            sA = pltpu.async_copy(hrowsA, OUT.at[dstb.at[0]], sem4, add=True)
            dB.wait()
            plsc.parallel_loop(0, K // 2, unroll=2)(mk_mbody(hrowsB, K // 2))
            sA.wait()
            pltpu.sync_copy(hrowsB, OUT.at[dstb.at[1]], add=True)
            return 0
        lax.fori_loop(0, NCH, p2_chunk, 0)
        plsc.subcore_barrier()

        for o, ln in _chunks(ROWS_PER_TILE, K):
            r = row0 + o
            pltpu.sync_copy(OUT.at[pl.ds(r, ln)], out_hbm.at[pl.ds(r, ln)])

    return sc_gat


_sc_gat1 = _make_sc_gat(8, C1, 8)
_sc_gat2 = _make_sc_gat(4, C2, 1)


def _att_mat(att, dc):
    H, _ = att.shape
    S = att[:, :, None] * jnp.eye(H, dtype=jnp.float32)[:, None, :]
    S = S.reshape(dc, H)
    return jnp.pad(S, ((0, 0), (0, 8 - H)))


def kernel(x, edge_index, W1, att_src1, att_dst1, b1, W2, att_src2, att_dst2,
           b2):
    xp = jnp.pad(x, ((0, N_PAD - N_NODES), (0, 0)))
    loops = jnp.arange(N_NODES, dtype=jnp.int32)
    fill = jnp.full((E_PAD - N_E,), N_NODES, jnp.int32)
    src = jnp.concatenate([edge_index[0].astype(jnp.int32), loops, fill])
    dst = jnp.concatenate([edge_index[1].astype(jnp.int32), loops, fill])
    src2d = src.reshape(E_PAD // K, 2, K // 2)
    dst2d = dst.reshape(E_PAD // K, 2, K // 2)

    Ss1 = _att_mat(att_src1, D_FEAT)
    Sd1 = _att_mat(att_dst1, D_FEAT)
    Ss2 = _att_mat(att_src2, C2)
    Sd2 = _att_mat(att_dst2, C2)

    h1, AS1, AD1 = _tc_proj(xp, W1, Ss1, Sd1)
    p1, _ = _sc_gat1(h1, AS1.reshape(-1), AD1.reshape(-1), src2d, dst2d)
    h2, AS2, AD2 = _tc_mid(p1, b1.reshape(1, -1), W2, Ss2, Sd2)
    p2, _ = _sc_gat2(h2, AS2.reshape(-1), AD2.reshape(-1), src2d, dst2d)
    b2p = jnp.pad(b2, (0, 128 - C2)).reshape(1, 128)
    out = _tc_final(p2, b2p)
    return out[:N_NODES, :C2]


# speedup vs baseline: 29.7874x; 1.1503x over previous
import functools

import jax
import jax.numpy as jnp
from jax import lax
from jax.experimental import pallas as pl
from jax.experimental.pallas import tpu as pltpu
from jax.experimental.pallas import tpu_sc as plsc

N_NODES = 10000
N_EDGES_IN = 320000
N_E = N_EDGES_IN + N_NODES
D_FEAT = 128
HEADS1 = 8
C1 = 16
C2 = 64

N_PAD = 10112
K = 128
E_PAD = 331776
ROWS_PER_TILE = N_PAD // 16
W_TILE = N_PAD * 8 // 16

TCB = 632


def _chunks(total, step):
    out = []
    o = 0
    while o < total:
        out.append((o, min(step, total - o)))
        o += step
    return out


def _tc_proj_body(x_ref, w_ref, ss_ref, sd_ref, h_ref, as_ref, ad_ref):
    h = jnp.dot(x_ref[...], w_ref[...], preferred_element_type=jnp.float32)
    h_ref[...] = h
    as_ref[...] = jnp.dot(h, ss_ref[...], preferred_element_type=jnp.float32)
    ad_ref[...] = jnp.dot(h, sd_ref[...], preferred_element_type=jnp.float32)


def _tc_proj(x, W, Ss, Sd):
    n, d = x.shape
    dc = W.shape[1]
    return pl.pallas_call(
        _tc_proj_body,
        grid=(n // TCB,),
        in_specs=[
            pl.BlockSpec((TCB, d), lambda i: (i, 0)),
            pl.BlockSpec((d, dc), lambda i: (0, 0)),
            pl.BlockSpec((dc, 8), lambda i: (0, 0)),
            pl.BlockSpec((dc, 8), lambda i: (0, 0)),
        ],
        out_specs=[
            pl.BlockSpec((TCB, dc), lambda i: (i, 0)),
            pl.BlockSpec((TCB, 8), lambda i: (i, 0)),
            pl.BlockSpec((TCB, 8), lambda i: (i, 0)),
        ],
        out_shape=[
            jax.ShapeDtypeStruct((n, dc), jnp.float32),
            jax.ShapeDtypeStruct((n, 8), jnp.float32),
            jax.ShapeDtypeStruct((n, 8), jnp.float32),
        ],
    )(x, W, Ss, Sd)


def _tc_mid_body(p_ref, b_ref, w_ref, ss_ref, sd_ref, h_ref, as_ref, ad_ref):
    hp = p_ref[...] + b_ref[...]
    hp = jnp.where(hp > 0, hp, jnp.exp(hp) - 1.0)
    h = jnp.dot(hp, w_ref[...], preferred_element_type=jnp.float32)
    h_ref[...] = jnp.concatenate(
        [h, jnp.zeros((h.shape[0], 128 - h.shape[1]), jnp.float32)], axis=1)
    as_ref[...] = jnp.dot(h, ss_ref[...], preferred_element_type=jnp.float32)
    ad_ref[...] = jnp.dot(h, sd_ref[...], preferred_element_type=jnp.float32)


def _tc_mid(p, b1, W2, Ss, Sd):
    n, d = p.shape
    dc = W2.shape[1]
    return pl.pallas_call(
        _tc_mid_body,
        grid=(n // TCB,),
        in_specs=[
            pl.BlockSpec((TCB, d), lambda i: (i, 0)),
            pl.BlockSpec((1, d), lambda i: (0, 0)),
            pl.BlockSpec((d, dc), lambda i: (0, 0)),
            pl.BlockSpec((dc, 8), lambda i: (0, 0)),
            pl.BlockSpec((dc, 8), lambda i: (0, 0)),
        ],
        out_specs=[
            pl.BlockSpec((TCB, 128), lambda i: (i, 0)),
            pl.BlockSpec((TCB, 8), lambda i: (i, 0)),
            pl.BlockSpec((TCB, 8), lambda i: (i, 0)),
        ],
        out_shape=[
            jax.ShapeDtypeStruct((n, 128), jnp.float32),
            jax.ShapeDtypeStruct((n, 8), jnp.float32),
            jax.ShapeDtypeStruct((n, 8), jnp.float32),
        ],
    )(p, b1, W2, Ss, Sd)


def _tc_final_body(p_ref, b_ref, o_ref):
    o_ref[...] = p_ref[...] + b_ref[...]


def _tc_final(p, b2):
    n, d = p.shape
    return pl.pallas_call(
        _tc_final_body,
        grid=(n // TCB,),
        in_specs=[
            pl.BlockSpec((TCB, d), lambda i: (i, 0)),
            pl.BlockSpec((1, d), lambda i: (0, 0)),
        ],
        out_specs=pl.BlockSpec((TCB, d), lambda i: (i, 0)),
        out_shape=jax.ShapeDtypeStruct((n, d), jnp.float32),
    )(p, b2)


def _make_sc_gat(NV, C, NH):
    E_TILE = E_PAD // 16
    NCH = E_TILE // K
    KB = K * NH
    NT = KB // K

    mesh = plsc.VectorSubcoreMesh(core_axis_name="c", subcore_axis_name="s",
                                  num_cores=1)

    @functools.partial(
        pl.kernel,
        out_type=[
            jax.ShapeDtypeStruct((N_PAD, 128), jnp.float32),
            jax.ShapeDtypeStruct((E_PAD * NH,), jnp.float32),
        ],
        mesh=mesh,
        compiler_params=pltpu.CompilerParams(needs_layout_passes=False),
        scratch_types=[
            pltpu.VMEM_SHARED((N_PAD * 8,), jnp.float32),
            pltpu.VMEM_SHARED((N_PAD * 8,), jnp.float32),
            pltpu.VMEM_SHARED((N_PAD * 8,), jnp.float32),
            pltpu.VMEM_SHARED((N_PAD, 128), jnp.float32),
            pltpu.VMEM((2, K // 2), jnp.int32),
            pltpu.VMEM((2, K // 2), jnp.int32),
            pltpu.VMEM((KB,), jnp.int32),
            pltpu.VMEM((KB,), jnp.int32),
            pltpu.VMEM((NT, K), jnp.int32),
            pltpu.VMEM((KB,), jnp.float32),
            pltpu.VMEM((KB,), jnp.float32),
            pltpu.VMEM((KB,), jnp.float32),
            pltpu.VMEM((KB,), jnp.float32),
            pltpu.VMEM((K // 2, 128), jnp.float32),
            pltpu.VMEM((K // 2, 128), jnp.float32),
            pltpu.SemaphoreType.DMA,
            pltpu.SemaphoreType.DMA,
            pltpu.SemaphoreType.DMA,
            pltpu.SemaphoreType.DMA,
        ],
    )
    def sc_gat(h_hbm, as_hbm, ad_hbm, src_hbm, dst_hbm, out_hbm, ee_hbm,
               ASF, ADF, DENF, OUT, srcb, dstb, eis, eid, eid2, asr, adr,
               eef, alphab, hrowsA, hrowsB, sem1, sem2, sem3, sem4):
        s = lax.axis_index("s")
        row0 = s * ROWS_PER_TILE
        iota = lax.iota(jnp.int32, 16)
        pr = iota >> 3
        pc = iota & 7
        z0 = iota * 0
        zv = jnp.zeros((16,), jnp.float32)

        def zrow(i, _):
            @pl.when(i < KB // 16)
            def _():
                eef[pl.ds(i * 16, 16)] = zv
            @pl.when(i < K // 2)
            def _():
                for v in range(8):
                    hrowsA[i, pl.ds(v * 16, 16)] = zv
                    hrowsB[i, pl.ds(v * 16, 16)] = zv
            return 0
        lax.fori_loop(0, K, zrow, 0)

        w0 = s * W_TILE
        for o, ln in _chunks(W_TILE, KB):
            pltpu.sync_copy(as_hbm.at[pl.ds(w0 + o, ln)], asr.at[pl.ds(0, ln)])
            pltpu.sync_copy(asr.at[pl.ds(0, ln)], ASF.at[pl.ds(w0 + o, ln)])
            pltpu.sync_copy(ad_hbm.at[pl.ds(w0 + o, ln)], adr.at[pl.ds(0, ln)])
            pltpu.sync_copy(adr.at[pl.ds(0, ln)], ADF.at[pl.ds(w0 + o, ln)])
            pltpu.sync_copy(eef.at[pl.ds(0, ln)], DENF.at[pl.ds(w0 + o, ln)])
        for o, ln in _chunks(ROWS_PER_TILE, K // 2):
            pltpu.sync_copy(hrowsA.at[pl.ds(0, ln)],
                            OUT.at[pl.ds(row0 + o, ln)])
        plsc.subcore_barrier()

        def p1_chunk(ch, _):
            off = s * E_TILE + ch * K
            di1 = pltpu.async_copy(src_hbm.at[off // K], srcb, sem1)
            di2 = pltpu.async_copy(dst_hbm.at[off // K], dstb, sem3)
            di1.wait()
            di2.wait()

            if NH == 8:
                def bidx(i):
                    k = i * 2
                    lane = pr + k
                    sv = plsc.load_gather(srcb, [lane >> 6, lane & 63])
                    dv = plsc.load_gather(dstb, [lane >> 6, lane & 63])
                    eis[pl.ds(i * 16, 16)] = sv * 8 + pc
                    ed = dv * 8 + pc
                    eid[pl.ds(i * 16, 16)] = ed
                    eid2[i >> 3, pl.ds((i & 7) * 16, 16)] = ed
                nb = K // 2
            else:
                def bidx(i):
                    lane = iota + i * 16
                    sv = plsc.load_gather(srcb, [lane >> 6, lane & 63])
                    dv = plsc.load_gather(dstb, [lane >> 6, lane & 63])
                    eis[pl.ds(i * 16, 16)] = sv * 8
                    ed = dv * 8
                    eid[pl.ds(i * 16, 16)] = ed
                    eid2[0, pl.ds(i * 16, 16)] = ed
                nb = K // 16
            plsc.parallel_loop(0, nb, unroll=4)(bidx)

            ds = []
            for j in range(NT):
                ds.append(pltpu.async_copy(
                    ASF.at[eis.at[pl.ds(j * K, K)]],
                    asr.at[pl.ds(j * K, K)], sem1))
                ds.append(pltpu.async_copy(
                    ADF.at[eid.at[pl.ds(j * K, K)]],
                    adr.at[pl.ds(j * K, K)], sem2))
            for d in ds:
                d.wait()

            def body(i):
                e = asr[pl.ds(i * 16, 16)] + adr[pl.ds(i * 16, 16)]
                e = jnp.maximum(e, 0.2 * e)
                eef[pl.ds(i * 16, 16)] = jnp.exp(e)
            plsc.parallel_loop(0, KB // 16, unroll=4)(body)

            ds2 = []
            for j in range(NT):
                ds2.append(pltpu.async_copy(
                    eef.at[pl.ds(j * K, K)], DENF.at[eid2.at[j]], sem2,
                    add=True))
            pltpu.sync_copy(eef, ee_hbm.at[pl.ds(off * NH, KB)])
            for d in ds2:
                d.wait()
            return 0
        lax.fori_loop(0, NCH, p1_chunk, 0)
        plsc.subcore_barrier()

        def p2_chunk(ch, _):
            off = s * E_TILE + ch * K
            di1 = pltpu.async_copy(src_hbm.at[off // K], srcb, sem1)
            di2 = pltpu.async_copy(dst_hbm.at[off // K], dstb, sem3)
            dee = pltpu.async_copy(ee_hbm.at[pl.ds(off * NH, KB)], eef, sem4)
            di1.wait()
            di2.wait()

            if NH == 8:
                def bidx(i):
                    lane = pr + i * 2
                    dv = plsc.load_gather(dstb, [lane >> 6, lane & 63])
                    eid[pl.ds(i * 16, 16)] = dv * 8 + pc
                nb = K // 2
            else:
                def bidx(i):
                    lane = iota + i * 16
                    dv = plsc.load_gather(dstb, [lane >> 6, lane & 63])
                    eid[pl.ds(i * 16, 16)] = dv * 8
                nb = K // 16
            plsc.parallel_loop(0, nb, unroll=4)(bidx)

            dA = pltpu.async_copy(h_hbm.at[srcb.at[0]], hrowsA, sem1)
            dB = pltpu.async_copy(h_hbm.at[srcb.at[1]], hrowsB, sem3)
            ds = []
            for j in range(NT):
                ds.append(pltpu.async_copy(
                    DENF.at[eid.at[pl.ds(j * K, K)]],
                    asr.at[pl.ds(j * K, K)], sem2))
            for d in ds:
                d.wait()
            dee.wait()

            def abody(i):
                ee = eef[pl.ds(i * 16, 16)]
                dn = asr[pl.ds(i * 16, 16)]
                alphab[pl.ds(i * 16, 16)] = ee / (dn + 1e-16)
            plsc.parallel_loop(0, KB // 16, unroll=4)(abody)

            def mk_mbody(buf, kofs):
                def mbody(k):
                    k8 = (k + kofs) * NH
                    for v in range(NV):
                        av = plsc.load_gather(
                            alphab,
                            [jnp.full((16,), k8 + (v * 16) // C, jnp.int32)])
                        hv = buf[k, pl.ds(v * 16, 16)]
                        buf[k, pl.ds(v * 16, 16)] = hv * av
                return mbody

            dA.wait()
            plsc.parallel_loop(0, K // 2, unroll=2)(mk_mbody(hrowsA, 0))
            sA = pltpu.async_copy(hrowsA, OUT.at[dstb.at[0]], sem4, add=True)
            dB.wait()
            plsc.parallel_loop(0, K // 2, unroll=2)(mk_mbody(hrowsB, K // 2))
            sA.wait()
            pltpu.sync_copy(hrowsB, OUT.at[dstb.at[1]], add=True)
            return 0
        lax.fori_loop(0, NCH, p2_chunk, 0)
        plsc.subcore_barrier()

        for o, ln in _chunks(ROWS_PER_TILE, K):
            r = row0 + o
            pltpu.sync_copy(OUT.at[pl.ds(r, ln)], out_hbm.at[pl.ds(r, ln)])

    return sc_gat


_sc_gat1 = _make_sc_gat(8, C1, 8)
_sc_gat2 = _make_sc_gat(4, C2, 1)


def _att_mat(att, dc):
    H, _ = att.shape
    S = att[:, :, None] * jnp.eye(H, dtype=jnp.float32)[:, None, :]
    S = S.reshape(dc, H)
    return jnp.pad(S, ((0, 0), (0, 8 - H)))


def kernel(x, edge_index, W1, att_src1, att_dst1, b1, W2, att_src2, att_dst2,
           b2):
    xp = jnp.pad(x, ((0, N_PAD - N_NODES), (0, 0)))
    loops = jnp.arange(N_NODES, dtype=jnp.int32)
    fill = jnp.full((E_PAD - N_E,), N_NODES, jnp.int32)
    src = jnp.concatenate([edge_index[0].astype(jnp.int32), loops, fill])
    dst = jnp.concatenate([edge_index[1].astype(jnp.int32), loops, fill])
    src2d = src.reshape(E_PAD // K, 2, K // 2)
    dst2d = dst.reshape(E_PAD // K, 2, K // 2)

    Ss1 = _att_mat(att_src1, D_FEAT)
    Sd1 = _att_mat(att_dst1, D_FEAT)
    Ss2 = _att_mat(att_src2, C2)
    Sd2 = _att_mat(att_dst2, C2)

    h1, AS1, AD1 = _tc_proj(xp, W1, Ss1, Sd1)
    p1, _ = _sc_gat1(h1, AS1.reshape(-1), AD1.reshape(-1), src2d, dst2d)
    h2, AS2, AD2 = _tc_mid(p1, b1.reshape(1, -1), W2, Ss2, Sd2)
    p2, _ = _sc_gat2(h2, AS2.reshape(-1), AD2.reshape(-1), src2d, dst2d)
    b2p = jnp.pad(b2, (0, 128 - C2)).reshape(1, 128)
    out = _tc_final(p2, b2p)
    return out[:N_NODES, :C2]


# speedup vs baseline: 32.2563x; 1.0829x over previous
import functools

import jax
import jax.numpy as jnp
from jax import lax
from jax.experimental import pallas as pl
from jax.experimental.pallas import tpu as pltpu
from jax.experimental.pallas import tpu_sc as plsc

N_NODES = 10000
N_EDGES_IN = 320000
N_E = N_EDGES_IN + N_NODES
D_FEAT = 128
HEADS1 = 8
C1 = 16
C2 = 64

N_PAD = 10112
K = 128
E_PAD = 331776
ROWS_PER_TILE = N_PAD // 16
W_TILE = N_PAD * 8 // 16

TCB = 632


def _chunks(total, step):
    out = []
    o = 0
    while o < total:
        out.append((o, min(step, total - o)))
        o += step
    return out


def _tc_proj_body(x_ref, w_ref, ss_ref, sd_ref, h_ref, as_ref, ad_ref):
    h = jnp.dot(x_ref[...], w_ref[...], preferred_element_type=jnp.float32)
    h_ref[...] = h
    as_ref[...] = jnp.dot(h, ss_ref[...], preferred_element_type=jnp.float32)
    ad_ref[...] = jnp.dot(h, sd_ref[...], preferred_element_type=jnp.float32)


def _tc_proj(x, W, Ss, Sd):
    n, d = x.shape
    dc = W.shape[1]
    return pl.pallas_call(
        _tc_proj_body,
        grid=(n // TCB,),
        in_specs=[
            pl.BlockSpec((TCB, d), lambda i: (i, 0)),
            pl.BlockSpec((d, dc), lambda i: (0, 0)),
            pl.BlockSpec((dc, 8), lambda i: (0, 0)),
            pl.BlockSpec((dc, 8), lambda i: (0, 0)),
        ],
        out_specs=[
            pl.BlockSpec((TCB, dc), lambda i: (i, 0)),
            pl.BlockSpec((TCB, 8), lambda i: (i, 0)),
            pl.BlockSpec((TCB, 8), lambda i: (i, 0)),
        ],
        out_shape=[
            jax.ShapeDtypeStruct((n, dc), jnp.float32),
            jax.ShapeDtypeStruct((n, 8), jnp.float32),
            jax.ShapeDtypeStruct((n, 8), jnp.float32),
        ],
    )(x, W, Ss, Sd)


def _tc_mid_body(p_ref, b_ref, w_ref, ss_ref, sd_ref, h_ref, as_ref, ad_ref):
    hp = p_ref[...] + b_ref[...]
    hp = jnp.where(hp > 0, hp, jnp.exp(hp) - 1.0)
    h = jnp.dot(hp, w_ref[...], preferred_element_type=jnp.float32)
    h_ref[...] = jnp.concatenate(
        [h, jnp.zeros((h.shape[0], 128 - h.shape[1]), jnp.float32)], axis=1)
    as_ref[...] = jnp.dot(h, ss_ref[...], preferred_element_type=jnp.float32)
    ad_ref[...] = jnp.dot(h, sd_ref[...], preferred_element_type=jnp.float32)


def _tc_mid(p, b1, W2, Ss, Sd):
    n, d = p.shape
    dc = W2.shape[1]
    return pl.pallas_call(
        _tc_mid_body,
        grid=(n // TCB,),
        in_specs=[
            pl.BlockSpec((TCB, d), lambda i: (i, 0)),
            pl.BlockSpec((1, d), lambda i: (0, 0)),
            pl.BlockSpec((d, dc), lambda i: (0, 0)),
            pl.BlockSpec((dc, 8), lambda i: (0, 0)),
            pl.BlockSpec((dc, 8), lambda i: (0, 0)),
        ],
        out_specs=[
            pl.BlockSpec((TCB, 128), lambda i: (i, 0)),
            pl.BlockSpec((TCB, 8), lambda i: (i, 0)),
            pl.BlockSpec((TCB, 8), lambda i: (i, 0)),
        ],
        out_shape=[
            jax.ShapeDtypeStruct((n, 128), jnp.float32),
            jax.ShapeDtypeStruct((n, 8), jnp.float32),
            jax.ShapeDtypeStruct((n, 8), jnp.float32),
        ],
    )(p, b1, W2, Ss, Sd)


def _tc_final_body(p_ref, b_ref, o_ref):
    o_ref[...] = p_ref[...] + b_ref[...]


def _tc_final(p, b2):
    n, d = p.shape
    return pl.pallas_call(
        _tc_final_body,
        grid=(n // TCB,),
        in_specs=[
            pl.BlockSpec((TCB, d), lambda i: (i, 0)),
            pl.BlockSpec((1, d), lambda i: (0, 0)),
        ],
        out_specs=pl.BlockSpec((TCB, d), lambda i: (i, 0)),
        out_shape=jax.ShapeDtypeStruct((n, d), jnp.float32),
    )(p, b2)


def _make_sc_gat(NV, C, NH):
    E_TILE = E_PAD // 16
    NCH = E_TILE // K
    KB = K * NH
    NT = KB // K

    mesh = plsc.VectorSubcoreMesh(core_axis_name="c", subcore_axis_name="s",
                                  num_cores=1)

    @functools.partial(
        pl.kernel,
        out_type=[
            jax.ShapeDtypeStruct((N_PAD, 128), jnp.float32),
            jax.ShapeDtypeStruct((E_PAD * NH,), jnp.float32),
        ],
        mesh=mesh,
        compiler_params=pltpu.CompilerParams(needs_layout_passes=False),
        scratch_types=[
            pltpu.VMEM_SHARED((N_PAD * 8,), jnp.float32),
            pltpu.VMEM_SHARED((N_PAD * 8,), jnp.float32),
            pltpu.VMEM_SHARED((N_PAD * 8,), jnp.float32),
            pltpu.VMEM_SHARED((N_PAD, 128), jnp.float32),
            pltpu.VMEM((2, K // 2), jnp.int32),
            pltpu.VMEM((2, K // 2), jnp.int32),
            pltpu.VMEM((KB,), jnp.int32),
            pltpu.VMEM((KB,), jnp.int32),
            pltpu.VMEM((NT, K), jnp.int32),
            pltpu.VMEM((KB,), jnp.float32),
            pltpu.VMEM((KB,), jnp.float32),
            pltpu.VMEM((KB,), jnp.float32),
            pltpu.VMEM((KB,), jnp.float32),
            pltpu.VMEM((K // 2, 128), jnp.float32),
            pltpu.VMEM((K // 2, 128), jnp.float32),
            pltpu.VMEM((2, K // 2), jnp.int32),
            pltpu.VMEM((2, K // 2), jnp.int32),
            pltpu.VMEM((KB,), jnp.int32),
            pltpu.VMEM((KB,), jnp.int32),
            pltpu.VMEM((NT, K), jnp.int32),
            pltpu.VMEM((KB,), jnp.float32),
            pltpu.VMEM((KB,), jnp.float32),
            pltpu.VMEM((KB,), jnp.float32),
            pltpu.SemaphoreType.DMA,
            pltpu.SemaphoreType.DMA,
            pltpu.SemaphoreType.DMA,
            pltpu.SemaphoreType.DMA,
            pltpu.SemaphoreType.DMA,
            pltpu.SemaphoreType.DMA,
        ],
    )
    def sc_gat(h_hbm, as_hbm, ad_hbm, src_hbm, dst_hbm, out_hbm, ee_hbm,
               ASF, ADF, DENF, OUT, srcb, dstb, eis, eid, eid2, asr, adr,
               eef, alphab, hrowsA, hrowsB, srcbB, dstbB, eisB, eidB, eid2B,
               asrB, adrB, eefB, sem1, sem2, sem3, sem4, sem5, sem6):
        s = lax.axis_index("s")
        row0 = s * ROWS_PER_TILE
        iota = lax.iota(jnp.int32, 16)
        pr = iota >> 3
        pc = iota & 7
        z0 = iota * 0
        zv = jnp.zeros((16,), jnp.float32)

        def zrow(i, _):
            @pl.when(i < KB // 16)
            def _():
                eef[pl.ds(i * 16, 16)] = zv
            @pl.when(i < K // 2)
            def _():
                for v in range(8):
                    hrowsA[i, pl.ds(v * 16, 16)] = zv
                    hrowsB[i, pl.ds(v * 16, 16)] = zv
            return 0
        lax.fori_loop(0, K, zrow, 0)

        w0 = s * W_TILE
        for o, ln in _chunks(W_TILE, KB):
            pltpu.sync_copy(as_hbm.at[pl.ds(w0 + o, ln)], asr.at[pl.ds(0, ln)])
            pltpu.sync_copy(asr.at[pl.ds(0, ln)], ASF.at[pl.ds(w0 + o, ln)])
            pltpu.sync_copy(ad_hbm.at[pl.ds(w0 + o, ln)], adr.at[pl.ds(0, ln)])
            pltpu.sync_copy(adr.at[pl.ds(0, ln)], ADF.at[pl.ds(w0 + o, ln)])
            pltpu.sync_copy(eef.at[pl.ds(0, ln)], DENF.at[pl.ds(w0 + o, ln)])
        for o, ln in _chunks(ROWS_PER_TILE, K // 2):
            pltpu.sync_copy(hrowsA.at[pl.ds(0, ln)],
                            OUT.at[pl.ds(row0 + o, ln)])
        plsc.subcore_barrier()

        setA = (srcb, dstb, eis, eid, eid2, asr, adr, eef, sem1, sem2)
        setB = (srcbB, dstbB, eisB, eidB, eid2B, asrB, adrB, eefB, sem3, sem4)

        def p1_fire(off, st):
            sb, db, ei_s, ei_d, ei_d2, a_s, a_d, ef, smA, smB = st
            di1 = pltpu.async_copy(src_hbm.at[off // K], sb, smA)
            di2 = pltpu.async_copy(dst_hbm.at[off // K], db, smB)
            di1.wait()
            di2.wait()

            if NH == 8:
                def bidx(i):
                    lane = pr + i * 2
                    sv = plsc.load_gather(sb, [lane >> 6, lane & 63])
                    dv = plsc.load_gather(db, [lane >> 6, lane & 63])
                    ei_s[pl.ds(i * 16, 16)] = sv * 8 + pc
                    ed = dv * 8 + pc
                    ei_d[pl.ds(i * 16, 16)] = ed
                    ei_d2[i >> 3, pl.ds((i & 7) * 16, 16)] = ed
                nb = K // 2
            else:
                def bidx(i):
                    lane = iota + i * 16
                    sv = plsc.load_gather(sb, [lane >> 6, lane & 63])
                    dv = plsc.load_gather(db, [lane >> 6, lane & 63])
                    ei_s[pl.ds(i * 16, 16)] = sv * 8
                    ed = dv * 8
                    ei_d[pl.ds(i * 16, 16)] = ed
                    ei_d2[0, pl.ds(i * 16, 16)] = ed
                nb = K // 16
            plsc.parallel_loop(0, nb, unroll=4)(bidx)

            ds = []
            for j in range(NT):
                ds.append(pltpu.async_copy(
                    ASF.at[ei_s.at[pl.ds(j * K, K)]],
                    a_s.at[pl.ds(j * K, K)], smA))
                ds.append(pltpu.async_copy(
                    ADF.at[ei_d.at[pl.ds(j * K, K)]],
                    a_d.at[pl.ds(j * K, K)], smB))
            return ds

        def p1_compute(off, st, ds):
            sb, db, ei_s, ei_d, ei_d2, a_s, a_d, ef, smA, smB = st
            for d in ds:
                d.wait()

            def body(i):
                e = a_s[pl.ds(i * 16, 16)] + a_d[pl.ds(i * 16, 16)]
                e = jnp.maximum(e, 0.2 * e)
                ef[pl.ds(i * 16, 16)] = jnp.exp(e)
            plsc.parallel_loop(0, KB // 16, unroll=4)(body)

            pend = []
            for j in range(NT):
                pend.append(pltpu.async_copy(
                    ef.at[pl.ds(j * K, K)], DENF.at[ei_d2.at[j]], sem5,
                    add=True))
            pend.append(pltpu.async_copy(
                ef, ee_hbm.at[pl.ds(off * NH, KB)], sem6))
            return pend

        def p1_pair(j2, _):
            offA = s * E_TILE + j2 * 2 * K
            offB = offA + K
            dsA = p1_fire(offA, setA)
            dsB = p1_fire(offB, setB)
            pA = p1_compute(offA, setA, dsA)
            pB = p1_compute(offB, setB, dsB)
            for d in pA + pB:
                d.wait()
            return 0
        lax.fori_loop(0, NCH // 2, p1_pair, 0)
        plsc.subcore_barrier()

        def p2_chunk(ch, _):
            off = s * E_TILE + ch * K
            di1 = pltpu.async_copy(src_hbm.at[off // K], srcb, sem1)
            di2 = pltpu.async_copy(dst_hbm.at[off // K], dstb, sem3)
            dee = pltpu.async_copy(ee_hbm.at[pl.ds(off * NH, KB)], eef, sem4)
            di1.wait()
            di2.wait()

            if NH == 8:
                def bidx(i):
                    lane = pr + i * 2
                    dv = plsc.load_gather(dstb, [lane >> 6, lane & 63])
                    eid[pl.ds(i * 16, 16)] = dv * 8 + pc
                nb = K // 2
            else:
                def bidx(i):
                    lane = iota + i * 16
                    dv = plsc.load_gather(dstb, [lane >> 6, lane & 63])
                    eid[pl.ds(i * 16, 16)] = dv * 8
                nb = K // 16
            plsc.parallel_loop(0, nb, unroll=4)(bidx)

            dA = pltpu.async_copy(h_hbm.at[srcb.at[0]], hrowsA, sem1)
            dB = pltpu.async_copy(h_hbm.at[srcb.at[1]], hrowsB, sem3)
            ds = []
            for j in range(NT):
                ds.append(pltpu.async_copy(
                    DENF.at[eid.at[pl.ds(j * K, K)]],
                    asr.at[pl.ds(j * K, K)], sem2))
            for d in ds:
                d.wait()
            dee.wait()

            def abody(i):
                ee = eef[pl.ds(i * 16, 16)]
                dn = asr[pl.ds(i * 16, 16)]
                alphab[pl.ds(i * 16, 16)] = ee / (dn + 1e-16)
            plsc.parallel_loop(0, KB // 16, unroll=4)(abody)

            def mk_mbody(buf, kofs):
                def mbody(k):
                    k8 = (k + kofs) * NH
                    for v in range(NV):
                        av = plsc.load_gather(
                            alphab,
                            [jnp.full((16,), k8 + (v * 16) // C, jnp.int32)])
                        hv = buf[k, pl.ds(v * 16, 16)]
                        buf[k, pl.ds(v * 16, 16)] = hv * av
                return mbody

            dA.wait()
            plsc.parallel_loop(0, K // 2, unroll=2)(mk_mbody(hrowsA, 0))
            sA = pltpu.async_copy(hrowsA, OUT.at[dstb.at[0]], sem4, add=True)
            dB.wait()
            plsc.parallel_loop(0, K // 2, unroll=2)(mk_mbody(hrowsB, K // 2))
            sA.wait()
            pltpu.sync_copy(hrowsB, OUT.at[dstb.at[1]], add=True)
            return 0
        lax.fori_loop(0, NCH, p2_chunk, 0)
        plsc.subcore_barrier()

        for o, ln in _chunks(ROWS_PER_TILE, K):
            r = row0 + o
            pltpu.sync_copy(OUT.at[pl.ds(r, ln)], out_hbm.at[pl.ds(r, ln)])

    return sc_gat


_sc_gat1 = _make_sc_gat(8, C1, 8)
_sc_gat2 = _make_sc_gat(4, C2, 1)


def _att_mat(att, dc):
    H, _ = att.shape
    S = att[:, :, None] * jnp.eye(H, dtype=jnp.float32)[:, None, :]
    S = S.reshape(dc, H)
    return jnp.pad(S, ((0, 0), (0, 8 - H)))


def kernel(x, edge_index, W1, att_src1, att_dst1, b1, W2, att_src2, att_dst2,
           b2):
    xp = jnp.pad(x, ((0, N_PAD - N_NODES), (0, 0)))
    loops = jnp.arange(N_NODES, dtype=jnp.int32)
    fill = jnp.full((E_PAD - N_E,), N_NODES, jnp.int32)
    src = jnp.concatenate([edge_index[0].astype(jnp.int32), loops, fill])
    dst = jnp.concatenate([edge_index[1].astype(jnp.int32), loops, fill])
    src2d = src.reshape(E_PAD // K, 2, K // 2)
    dst2d = dst.reshape(E_PAD // K, 2, K // 2)

    Ss1 = _att_mat(att_src1, D_FEAT)
    Sd1 = _att_mat(att_dst1, D_FEAT)
    Ss2 = _att_mat(att_src2, C2)
    Sd2 = _att_mat(att_dst2, C2)

    h1, AS1, AD1 = _tc_proj(xp, W1, Ss1, Sd1)
    p1, _ = _sc_gat1(h1, AS1.reshape(-1), AD1.reshape(-1), src2d, dst2d)
    h2, AS2, AD2 = _tc_mid(p1, b1.reshape(1, -1), W2, Ss2, Sd2)
    p2, _ = _sc_gat2(h2, AS2.reshape(-1), AD2.reshape(-1), src2d, dst2d)
    b2p = jnp.pad(b2, (0, 128 - C2)).reshape(1, 128)
    out = _tc_final(p2, b2p)
    return out[:N_NODES, :C2]


# speedup vs baseline: 32.3321x; 1.0024x over previous
import functools

import jax
import jax.numpy as jnp
from jax import lax
from jax.experimental import pallas as pl
from jax.experimental.pallas import tpu as pltpu
from jax.experimental.pallas import tpu_sc as plsc

N_NODES = 10000
N_EDGES_IN = 320000
N_E = N_EDGES_IN + N_NODES
D_FEAT = 128
HEADS1 = 8
C1 = 16
C2 = 64

N_PAD = 10112
K = 128
E_PAD = 331776
ROWS_PER_TILE = N_PAD // 16
W_TILE = N_PAD * 8 // 16

TCB = 632


def _chunks(total, step):
    out = []
    o = 0
    while o < total:
        out.append((o, min(step, total - o)))
        o += step
    return out


def _tc_proj_body(x_ref, w_ref, ss_ref, sd_ref, h_ref, as_ref, ad_ref):
    h = jnp.dot(x_ref[...], w_ref[...], preferred_element_type=jnp.float32)
    h_ref[...] = h
    as_ref[...] = jnp.dot(h, ss_ref[...], preferred_element_type=jnp.float32)
    ad_ref[...] = jnp.dot(h, sd_ref[...], preferred_element_type=jnp.float32)


def _tc_proj(x, W, Ss, Sd):
    n, d = x.shape
    dc = W.shape[1]
    return pl.pallas_call(
        _tc_proj_body,
        grid=(n // TCB,),
        in_specs=[
            pl.BlockSpec((TCB, d), lambda i: (i, 0)),
            pl.BlockSpec((d, dc), lambda i: (0, 0)),
            pl.BlockSpec((dc, 8), lambda i: (0, 0)),
            pl.BlockSpec((dc, 8), lambda i: (0, 0)),
        ],
        out_specs=[
            pl.BlockSpec((TCB, dc), lambda i: (i, 0)),
            pl.BlockSpec((TCB, 8), lambda i: (i, 0)),
            pl.BlockSpec((TCB, 8), lambda i: (i, 0)),
        ],
        out_shape=[
            jax.ShapeDtypeStruct((n, dc), jnp.float32),
            jax.ShapeDtypeStruct((n, 8), jnp.float32),
            jax.ShapeDtypeStruct((n, 8), jnp.float32),
        ],
    )(x, W, Ss, Sd)


def _tc_mid_body(p_ref, b_ref, w_ref, ss_ref, sd_ref, h_ref, as_ref, ad_ref):
    hp = p_ref[...] + b_ref[...]
    hp = jnp.where(hp > 0, hp, jnp.exp(hp) - 1.0)
    h = jnp.dot(hp, w_ref[...], preferred_element_type=jnp.float32)
    h_ref[...] = jnp.concatenate(
        [h, jnp.zeros((h.shape[0], 128 - h.shape[1]), jnp.float32)], axis=1)
    as_ref[...] = jnp.dot(h, ss_ref[...], preferred_element_type=jnp.float32)
    ad_ref[...] = jnp.dot(h, sd_ref[...], preferred_element_type=jnp.float32)


def _tc_mid(p, b1, W2, Ss, Sd):
    n, d = p.shape
    dc = W2.shape[1]
    return pl.pallas_call(
        _tc_mid_body,
        grid=(n // TCB,),
        in_specs=[
            pl.BlockSpec((TCB, d), lambda i: (i, 0)),
            pl.BlockSpec((1, d), lambda i: (0, 0)),
            pl.BlockSpec((d, dc), lambda i: (0, 0)),
            pl.BlockSpec((dc, 8), lambda i: (0, 0)),
            pl.BlockSpec((dc, 8), lambda i: (0, 0)),
        ],
        out_specs=[
            pl.BlockSpec((TCB, 128), lambda i: (i, 0)),
            pl.BlockSpec((TCB, 8), lambda i: (i, 0)),
            pl.BlockSpec((TCB, 8), lambda i: (i, 0)),
        ],
        out_shape=[
            jax.ShapeDtypeStruct((n, 128), jnp.float32),
            jax.ShapeDtypeStruct((n, 8), jnp.float32),
            jax.ShapeDtypeStruct((n, 8), jnp.float32),
        ],
    )(p, b1, W2, Ss, Sd)


def _tc_final_body(p_ref, b_ref, o_ref):
    o_ref[...] = p_ref[...] + b_ref[...]


def _tc_final(p, b2):
    n, d = p.shape
    return pl.pallas_call(
        _tc_final_body,
        grid=(n // TCB,),
        in_specs=[
            pl.BlockSpec((TCB, d), lambda i: (i, 0)),
            pl.BlockSpec((1, d), lambda i: (0, 0)),
        ],
        out_specs=pl.BlockSpec((TCB, d), lambda i: (i, 0)),
        out_shape=jax.ShapeDtypeStruct((n, d), jnp.float32),
    )(p, b2)


def _make_sc_gat(NV, C, NH):
    E_TILE = E_PAD // 16
    NCH = E_TILE // K
    KB = K * NH
    NT = KB // K

    mesh = plsc.VectorSubcoreMesh(core_axis_name="c", subcore_axis_name="s",
                                  num_cores=1)

    @functools.partial(
        pl.kernel,
        out_type=[
            jax.ShapeDtypeStruct((N_PAD, 128), jnp.float32),
            jax.ShapeDtypeStruct((E_PAD * NH,), jnp.float32),
        ],
        mesh=mesh,
        compiler_params=pltpu.CompilerParams(needs_layout_passes=False),
        scratch_types=[
            pltpu.VMEM_SHARED((N_PAD * 8,), jnp.float32),
            pltpu.VMEM_SHARED((N_PAD * 8,), jnp.float32),
            pltpu.VMEM_SHARED((N_PAD * 8,), jnp.float32),
            pltpu.VMEM_SHARED((N_PAD, 128), jnp.float32),
            pltpu.VMEM((2, K // 2), jnp.int32),
            pltpu.VMEM((2, K // 2), jnp.int32),
            pltpu.VMEM((KB,), jnp.int32),
            pltpu.VMEM((KB,), jnp.int32),
            pltpu.VMEM((NT, K), jnp.int32),
            pltpu.VMEM((KB,), jnp.float32),
            pltpu.VMEM((KB,), jnp.float32),
            pltpu.VMEM((KB,), jnp.float32),
            pltpu.VMEM((KB,), jnp.float32),
            pltpu.VMEM((K // 2, 128), jnp.float32),
            pltpu.VMEM((K // 2, 128), jnp.float32),
            pltpu.VMEM((2, K // 2), jnp.int32),
            pltpu.VMEM((2, K // 2), jnp.int32),
            pltpu.VMEM((KB,), jnp.int32),
            pltpu.VMEM((KB,), jnp.int32),
            pltpu.VMEM((NT, K), jnp.int32),
            pltpu.VMEM((KB,), jnp.float32),
            pltpu.VMEM((KB,), jnp.float32),
            pltpu.VMEM((KB,), jnp.float32),
            pltpu.SemaphoreType.DMA,
            pltpu.SemaphoreType.DMA,
            pltpu.SemaphoreType.DMA,
            pltpu.SemaphoreType.DMA,
            pltpu.SemaphoreType.DMA,
            pltpu.SemaphoreType.DMA,
        ],
    )
    def sc_gat(h_hbm, as_hbm, ad_hbm, src_hbm, dst_hbm, out_hbm, ee_hbm,
               ASF, ADF, DENF, OUT, srcb, dstb, eis, eid, eid2, asr, adr,
               eef, alphab, hrowsA, hrowsB, srcbB, dstbB, eisB, eidB, eid2B,
               asrB, adrB, eefB, sem1, sem2, sem3, sem4, sem5, sem6):
        s = lax.axis_index("s")
        row0 = s * ROWS_PER_TILE
        iota = lax.iota(jnp.int32, 16)
        pr = iota >> 3
        pc = iota & 7
        z0 = iota * 0
        zv = jnp.zeros((16,), jnp.float32)

        def zrow(i, _):
            @pl.when(i < KB // 16)
            def _():
                eef[pl.ds(i * 16, 16)] = zv
            @pl.when(i < K // 2)
            def _():
                for v in range(8):
                    hrowsA[i, pl.ds(v * 16, 16)] = zv
                    hrowsB[i, pl.ds(v * 16, 16)] = zv
            return 0
        lax.fori_loop(0, K, zrow, 0)

        w0 = s * W_TILE
        for o, ln in _chunks(W_TILE, KB):
            pltpu.sync_copy(as_hbm.at[pl.ds(w0 + o, ln)], asr.at[pl.ds(0, ln)])
            pltpu.sync_copy(asr.at[pl.ds(0, ln)], ASF.at[pl.ds(w0 + o, ln)])
            pltpu.sync_copy(ad_hbm.at[pl.ds(w0 + o, ln)], adr.at[pl.ds(0, ln)])
            pltpu.sync_copy(adr.at[pl.ds(0, ln)], ADF.at[pl.ds(w0 + o, ln)])
            pltpu.sync_copy(eef.at[pl.ds(0, ln)], DENF.at[pl.ds(w0 + o, ln)])
        for o, ln in _chunks(ROWS_PER_TILE, K // 2):
            pltpu.sync_copy(hrowsA.at[pl.ds(0, ln)],
                            OUT.at[pl.ds(row0 + o, ln)])
        plsc.subcore_barrier()

        setA = (srcb, dstb, eis, eid, eid2, asr, adr, eef, sem1, sem2)
        setB = (srcbB, dstbB, eisB, eidB, eid2B, asrB, adrB, eefB, sem3, sem4)

        def p1_fire(off, st):
            sb, db, ei_s, ei_d, ei_d2, a_s, a_d, ef, smA, smB = st
            di1 = pltpu.async_copy(src_hbm.at[off // K], sb, smA)
            di2 = pltpu.async_copy(dst_hbm.at[off // K], db, smB)
            di1.wait()
            di2.wait()

            if NH == 8:
                def bidx(i):
                    lane = pr + i * 2
                    sv = plsc.load_gather(sb, [lane >> 6, lane & 63])
                    dv = plsc.load_gather(db, [lane >> 6, lane & 63])
                    ei_s[pl.ds(i * 16, 16)] = sv * 8 + pc
                    ed = dv * 8 + pc
                    ei_d[pl.ds(i * 16, 16)] = ed
                    ei_d2[i >> 3, pl.ds((i & 7) * 16, 16)] = ed
                nb = K // 2
            else:
                def bidx(i):
                    lane = iota + i * 16
                    sv = plsc.load_gather(sb, [lane >> 6, lane & 63])
                    dv = plsc.load_gather(db, [lane >> 6, lane & 63])
                    ei_s[pl.ds(i * 16, 16)] = sv * 8
                    ed = dv * 8
                    ei_d[pl.ds(i * 16, 16)] = ed
                    ei_d2[0, pl.ds(i * 16, 16)] = ed
                nb = K // 16
            plsc.parallel_loop(0, nb, unroll=4)(bidx)

            ds = []
            for j in range(NT):
                ds.append(pltpu.async_copy(
                    ASF.at[ei_s.at[pl.ds(j * K, K)]],
                    a_s.at[pl.ds(j * K, K)], smA))
                ds.append(pltpu.async_copy(
                    ADF.at[ei_d.at[pl.ds(j * K, K)]],
                    a_d.at[pl.ds(j * K, K)], smB))
            return ds

        def p1_compute(off, st, ds):
            sb, db, ei_s, ei_d, ei_d2, a_s, a_d, ef, smA, smB = st
            for d in ds:
                d.wait()

            def body(i):
                e = a_s[pl.ds(i * 16, 16)] + a_d[pl.ds(i * 16, 16)]
                e = jnp.maximum(e, 0.2 * e)
                ef[pl.ds(i * 16, 16)] = jnp.exp(e)
            plsc.parallel_loop(0, KB // 16, unroll=4)(body)

            pend = []
            for j in range(NT):
                pend.append(pltpu.async_copy(
                    ef.at[pl.ds(j * K, K)], DENF.at[ei_d2.at[j]], sem5,
                    add=True))
            pend.append(pltpu.async_copy(
                ef, ee_hbm.at[pl.ds(off * NH, KB)], sem6))
            return pend

        def p1_pair(j2, _):
            offA = s * E_TILE + j2 * 2 * K
            offB = offA + K
            dsA = p1_fire(offA, setA)
            dsB = p1_fire(offB, setB)
            pA = p1_compute(offA, setA, dsA)
            pB = p1_compute(offB, setB, dsB)
            for d in pA + pB:
                d.wait()
            return 0
        lax.fori_loop(0, NCH // 2, p1_pair, 0)
        plsc.subcore_barrier()

        def p2_chunk(ch, _):
            off = s * E_TILE + ch * K
            di1 = pltpu.async_copy(src_hbm.at[off // K], srcb, sem1)
            di2 = pltpu.async_copy(dst_hbm.at[off // K], dstb, sem3)
            dee = pltpu.async_copy(ee_hbm.at[pl.ds(off * NH, KB)], eef, sem4)
            di1.wait()
            di2.wait()

            if NH == 8:
                def bidx(i):
                    lane = pr + i * 2
                    dv = plsc.load_gather(dstb, [lane >> 6, lane & 63])
                    eid[pl.ds(i * 16, 16)] = dv * 8 + pc
                nb = K // 2
            else:
                def bidx(i):
                    lane = iota + i * 16
                    dv = plsc.load_gather(dstb, [lane >> 6, lane & 63])
                    eid[pl.ds(i * 16, 16)] = dv * 8
                nb = K // 16
            plsc.parallel_loop(0, nb, unroll=4)(bidx)

            dA = pltpu.async_copy(h_hbm.at[srcb.at[0]], hrowsA, sem1)
            dB = pltpu.async_copy(h_hbm.at[srcb.at[1]], hrowsB, sem3)
            ds = []
            for j in range(NT):
                ds.append(pltpu.async_copy(
                    DENF.at[eid.at[pl.ds(j * K, K)]],
                    asr.at[pl.ds(j * K, K)], sem2))
            for d in ds:
                d.wait()
            dee.wait()

            def abody(i):
                ee = eef[pl.ds(i * 16, 16)]
                dn = asr[pl.ds(i * 16, 16)]
                alphab[pl.ds(i * 16, 16)] = ee / (dn + 1e-16)
            plsc.parallel_loop(0, KB // 16, unroll=4)(abody)

            def mk_mbody(buf, kofs):
                def mbody(k):
                    k8 = (k + kofs) * NH
                    for v in range(NV):
                        av = plsc.load_gather(
                            alphab,
                            [jnp.full((16,), k8 + (v * 16) // C, jnp.int32)])
                        hv = buf[k, pl.ds(v * 16, 16)]
                        buf[k, pl.ds(v * 16, 16)] = hv * av
                return mbody

            dA.wait()
            plsc.parallel_loop(0, K // 2, unroll=4)(mk_mbody(hrowsA, 0))
            sA = pltpu.async_copy(hrowsA, OUT.at[dstb.at[0]], sem4, add=True)
            dB.wait()
            plsc.parallel_loop(0, K // 2, unroll=4)(mk_mbody(hrowsB, K // 2))
            sA.wait()
            pltpu.sync_copy(hrowsB, OUT.at[dstb.at[1]], add=True)
            return 0
        lax.fori_loop(0, NCH, p2_chunk, 0)
        plsc.subcore_barrier()

        for o, ln in _chunks(ROWS_PER_TILE, K):
            r = row0 + o
            pltpu.sync_copy(OUT.at[pl.ds(r, ln)], out_hbm.at[pl.ds(r, ln)])

    return sc_gat


_sc_gat1 = _make_sc_gat(8, C1, 8)
_sc_gat2 = _make_sc_gat(4, C2, 1)


def _att_mat(att, dc):
    H, _ = att.shape
    S = att[:, :, None] * jnp.eye(H, dtype=jnp.float32)[:, None, :]
    S = S.reshape(dc, H)
    return jnp.pad(S, ((0, 0), (0, 8 - H)))


def kernel(x, edge_index, W1, att_src1, att_dst1, b1, W2, att_src2, att_dst2,
           b2):
    xp = jnp.pad(x, ((0, N_PAD - N_NODES), (0, 0)))
    loops = jnp.arange(N_NODES, dtype=jnp.int32)
    fill = jnp.full((E_PAD - N_E,), N_NODES, jnp.int32)
    src = jnp.concatenate([edge_index[0].astype(jnp.int32), loops, fill])
    dst = jnp.concatenate([edge_index[1].astype(jnp.int32), loops, fill])
    src2d = src.reshape(E_PAD // K, 2, K // 2)
    dst2d = dst.reshape(E_PAD // K, 2, K // 2)

    Ss1 = _att_mat(att_src1, D_FEAT)
    Sd1 = _att_mat(att_dst1, D_FEAT)
    Ss2 = _att_mat(att_src2, C2)
    Sd2 = _att_mat(att_dst2, C2)

    h1, AS1, AD1 = _tc_proj(xp, W1, Ss1, Sd1)
    p1, _ = _sc_gat1(h1, AS1.reshape(-1), AD1.reshape(-1), src2d, dst2d)
    h2, AS2, AD2 = _tc_mid(p1, b1.reshape(1, -1), W2, Ss2, Sd2)
    p2, _ = _sc_gat2(h2, AS2.reshape(-1), AD2.reshape(-1), src2d, dst2d)
    b2p = jnp.pad(b2, (0, 128 - C2)).reshape(1, 128)
    out = _tc_final(p2, b2p)
    return out[:N_NODES, :C2]


# speedup vs baseline: 35.9417x; 1.1116x over previous
import functools

import jax
import jax.numpy as jnp
from jax import lax
from jax.experimental import pallas as pl
from jax.experimental.pallas import tpu as pltpu
from jax.experimental.pallas import tpu_sc as plsc

N_NODES = 10000
N_EDGES_IN = 320000
N_E = N_EDGES_IN + N_NODES
D_FEAT = 128
HEADS1 = 8
C1 = 16
C2 = 64

N_PAD = 10112
K = 128
E_PAD = 331776
ROWS_PER_TILE = N_PAD // 16
W_TILE = N_PAD * 8 // 16

TCB = 632


def _chunks(total, step):
    out = []
    o = 0
    while o < total:
        out.append((o, min(step, total - o)))
        o += step
    return out


def _tc_proj_body(x_ref, w_ref, ss_ref, sd_ref, h_ref, as_ref, ad_ref):
    h = jnp.dot(x_ref[...], w_ref[...], preferred_element_type=jnp.float32)
    h_ref[...] = h
    as_ref[...] = jnp.dot(h, ss_ref[...], preferred_element_type=jnp.float32)
    ad_ref[...] = jnp.dot(h, sd_ref[...], preferred_element_type=jnp.float32)


def _tc_proj(x, W, Ss, Sd):
    n, d = x.shape
    dc = W.shape[1]
    return pl.pallas_call(
        _tc_proj_body,
        grid=(n // TCB,),
        in_specs=[
            pl.BlockSpec((TCB, d), lambda i: (i, 0)),
            pl.BlockSpec((d, dc), lambda i: (0, 0)),
            pl.BlockSpec((dc, 8), lambda i: (0, 0)),
            pl.BlockSpec((dc, 8), lambda i: (0, 0)),
        ],
        out_specs=[
            pl.BlockSpec((TCB, dc), lambda i: (i, 0)),
            pl.BlockSpec((TCB, 8), lambda i: (i, 0)),
            pl.BlockSpec((TCB, 8), lambda i: (i, 0)),
        ],
        out_shape=[
            jax.ShapeDtypeStruct((n, dc), jnp.float32),
            jax.ShapeDtypeStruct((n, 8), jnp.float32),
            jax.ShapeDtypeStruct((n, 8), jnp.float32),
        ],
    )(x, W, Ss, Sd)


def _tc_mid_body(p_ref, b_ref, w_ref, ss_ref, sd_ref, h_ref, as_ref, ad_ref):
    hp = p_ref[...] + b_ref[...]
    hp = jnp.where(hp > 0, hp, jnp.exp(hp) - 1.0)
    h = jnp.dot(hp, w_ref[...], preferred_element_type=jnp.float32)
    h_ref[...] = jnp.concatenate(
        [h, jnp.zeros((h.shape[0], 128 - h.shape[1]), jnp.float32)], axis=1)
    as_ref[...] = jnp.dot(h, ss_ref[...], preferred_element_type=jnp.float32)
    ad_ref[...] = jnp.dot(h, sd_ref[...], preferred_element_type=jnp.float32)


def _tc_mid(p, b1, W2, Ss, Sd):
    n, d = p.shape
    dc = W2.shape[1]
    return pl.pallas_call(
        _tc_mid_body,
        grid=(n // TCB,),
        in_specs=[
            pl.BlockSpec((TCB, d), lambda i: (i, 0)),
            pl.BlockSpec((1, d), lambda i: (0, 0)),
            pl.BlockSpec((d, dc), lambda i: (0, 0)),
            pl.BlockSpec((dc, 8), lambda i: (0, 0)),
            pl.BlockSpec((dc, 8), lambda i: (0, 0)),
        ],
        out_specs=[
            pl.BlockSpec((TCB, 128), lambda i: (i, 0)),
            pl.BlockSpec((TCB, 8), lambda i: (i, 0)),
            pl.BlockSpec((TCB, 8), lambda i: (i, 0)),
        ],
        out_shape=[
            jax.ShapeDtypeStruct((n, 128), jnp.float32),
            jax.ShapeDtypeStruct((n, 8), jnp.float32),
            jax.ShapeDtypeStruct((n, 8), jnp.float32),
        ],
    )(p, b1, W2, Ss, Sd)


def _tc_final_body(p_ref, b_ref, o_ref):
    o_ref[...] = p_ref[...] + b_ref[...]


def _tc_final(p, b2):
    n, d = p.shape
    return pl.pallas_call(
        _tc_final_body,
        grid=(n // TCB,),
        in_specs=[
            pl.BlockSpec((TCB, d), lambda i: (i, 0)),
            pl.BlockSpec((1, d), lambda i: (0, 0)),
        ],
        out_specs=pl.BlockSpec((TCB, d), lambda i: (i, 0)),
        out_shape=jax.ShapeDtypeStruct((n, d), jnp.float32),
    )(p, b2)


def _make_sc_gat(NV, C, NH):
    E_TILE = E_PAD // 16
    NCH = E_TILE // K
    KB = K * NH
    NT = KB // K

    mesh = plsc.VectorSubcoreMesh(core_axis_name="c", subcore_axis_name="s",
                                  num_cores=1)

    @functools.partial(
        pl.kernel,
        out_type=[
            jax.ShapeDtypeStruct((N_PAD, 128), jnp.float32),
            jax.ShapeDtypeStruct((E_PAD * NH,), jnp.float32),
        ],
        mesh=mesh,
        compiler_params=pltpu.CompilerParams(needs_layout_passes=False),
        scratch_types=[
            pltpu.VMEM_SHARED((N_PAD * 8,), jnp.float32),
            pltpu.VMEM_SHARED((N_PAD * 8,), jnp.float32),
            pltpu.VMEM_SHARED((N_PAD * 8,), jnp.float32),
            pltpu.VMEM_SHARED((N_PAD, 128), jnp.float32),
            pltpu.VMEM((2, K // 2), jnp.int32),
            pltpu.VMEM((2, K // 2), jnp.int32),
            pltpu.VMEM((KB,), jnp.int32),
            pltpu.VMEM((KB,), jnp.int32),
            pltpu.VMEM((NT, K), jnp.int32),
            pltpu.VMEM((KB,), jnp.float32),
            pltpu.VMEM((KB,), jnp.float32),
            pltpu.VMEM((KB,), jnp.float32),
            pltpu.VMEM((KB,), jnp.float32),
            pltpu.VMEM((K // 2, 128), jnp.float32),
            pltpu.VMEM((K // 2, 128), jnp.float32),
            pltpu.VMEM((2, K // 2), jnp.int32),
            pltpu.VMEM((2, K // 2), jnp.int32),
            pltpu.VMEM((KB,), jnp.int32),
            pltpu.VMEM((KB,), jnp.int32),
            pltpu.VMEM((NT, K), jnp.int32),
            pltpu.VMEM((KB,), jnp.float32),
            pltpu.VMEM((KB,), jnp.float32),
            pltpu.VMEM((KB,), jnp.float32),
            pltpu.SemaphoreType.DMA,
            pltpu.SemaphoreType.DMA,
            pltpu.SemaphoreType.DMA,
            pltpu.SemaphoreType.DMA,
            pltpu.SemaphoreType.DMA,
            pltpu.SemaphoreType.DMA,
            pltpu.SemaphoreType.DMA,
            pltpu.SemaphoreType.DMA,
            pltpu.SemaphoreType.DMA,
            pltpu.SemaphoreType.DMA,
        ],
    )
    def sc_gat(h_hbm, as_hbm, ad_hbm, src_hbm, dst_hbm, out_hbm, ee_hbm,
               ASF, ADF, DENF, OUT, srcb, dstb, eis, eid, eid2, asr, adr,
               eef, alphab, hrowsA, hrowsB, srcbB, dstbB, eisB, eidB, eid2B,
               asrB, adrB, eefB, sem1, sem2, sem3, sem4, sem5, sem6, sem7,
               sem8, sem9, sem10):
        s = lax.axis_index("s")
        row0 = s * ROWS_PER_TILE
        iota = lax.iota(jnp.int32, 16)
        pr = iota >> 3
        pc = iota & 7
        z0 = iota * 0
        zv = jnp.zeros((16,), jnp.float32)

        def zrow(i, _):
            @pl.when(i < KB // 16)
            def _():
                eef[pl.ds(i * 16, 16)] = zv
            @pl.when(i < K // 2)
            def _():
                for v in range(8):
                    hrowsA[i, pl.ds(v * 16, 16)] = zv
                    hrowsB[i, pl.ds(v * 16, 16)] = zv
            return 0
        lax.fori_loop(0, K, zrow, 0)

        w0 = s * W_TILE
        for o, ln in _chunks(W_TILE, KB):
            pltpu.sync_copy(as_hbm.at[pl.ds(w0 + o, ln)], asr.at[pl.ds(0, ln)])
            pltpu.sync_copy(asr.at[pl.ds(0, ln)], ASF.at[pl.ds(w0 + o, ln)])
            pltpu.sync_copy(ad_hbm.at[pl.ds(w0 + o, ln)], adr.at[pl.ds(0, ln)])
            pltpu.sync_copy(adr.at[pl.ds(0, ln)], ADF.at[pl.ds(w0 + o, ln)])
            pltpu.sync_copy(eef.at[pl.ds(0, ln)], DENF.at[pl.ds(w0 + o, ln)])
        for o, ln in _chunks(ROWS_PER_TILE, K // 2):
            pltpu.sync_copy(hrowsA.at[pl.ds(0, ln)],
                            OUT.at[pl.ds(row0 + o, ln)])
        plsc.subcore_barrier()

        setA = (srcb, dstb, eis, eid, eid2, asr, adr, eef, sem1, sem2)
        setB = (srcbB, dstbB, eisB, eidB, eid2B, asrB, adrB, eefB, sem3, sem4)

        def p1_fire(off, st):
            sb, db, ei_s, ei_d, ei_d2, a_s, a_d, ef, smA, smB = st
            di1 = pltpu.async_copy(src_hbm.at[off // K], sb, smA)
            di2 = pltpu.async_copy(dst_hbm.at[off // K], db, smB)
            di1.wait()
            di2.wait()

            if NH == 8:
                def bidx(i):
                    lane = pr + i * 2
                    sv = plsc.load_gather(sb, [lane >> 6, lane & 63])
                    dv = plsc.load_gather(db, [lane >> 6, lane & 63])
                    ei_s[pl.ds(i * 16, 16)] = sv * 8 + pc
                    ed = dv * 8 + pc
                    ei_d[pl.ds(i * 16, 16)] = ed
                    ei_d2[i >> 3, pl.ds((i & 7) * 16, 16)] = ed
                nb = K // 2
            else:
                def bidx(i):
                    lane = iota + i * 16
                    sv = plsc.load_gather(sb, [lane >> 6, lane & 63])
                    dv = plsc.load_gather(db, [lane >> 6, lane & 63])
                    ei_s[pl.ds(i * 16, 16)] = sv * 8
                    ed = dv * 8
                    ei_d[pl.ds(i * 16, 16)] = ed
                    ei_d2[0, pl.ds(i * 16, 16)] = ed
                nb = K // 16
            plsc.parallel_loop(0, nb, unroll=4)(bidx)

            ds = []
            for j in range(NT):
                ds.append(pltpu.async_copy(
                    ASF.at[ei_s.at[pl.ds(j * K, K)]],
                    a_s.at[pl.ds(j * K, K)], smA))
                ds.append(pltpu.async_copy(
                    ADF.at[ei_d.at[pl.ds(j * K, K)]],
                    a_d.at[pl.ds(j * K, K)], smB))
            return ds

        def p1_compute(off, st, ds):
            sb, db, ei_s, ei_d, ei_d2, a_s, a_d, ef, smA, smB = st
            for d in ds:
                d.wait()

            def body(i):
                e = a_s[pl.ds(i * 16, 16)] + a_d[pl.ds(i * 16, 16)]
                e = jnp.maximum(e, 0.2 * e)
                ef[pl.ds(i * 16, 16)] = jnp.exp(e)
            plsc.parallel_loop(0, KB // 16, unroll=4)(body)

            pend = []
            for j in range(NT):
                pend.append(pltpu.async_copy(
                    ef.at[pl.ds(j * K, K)], DENF.at[ei_d2.at[j]], sem5,
                    add=True))
            pend.append(pltpu.async_copy(
                ef, ee_hbm.at[pl.ds(off * NH, KB)], sem6))
            return pend

        def p1_pair(j2, _):
            offA = s * E_TILE + j2 * 2 * K
            offB = offA + K
            dsA = p1_fire(offA, setA)
            dsB = p1_fire(offB, setB)
            pA = p1_compute(offA, setA, dsA)
            pB = p1_compute(offB, setB, dsB)
            for d in pA + pB:
                d.wait()
            return 0
        lax.fori_loop(0, NCH // 2, p1_pair, 0)
        plsc.subcore_barrier()

        def p2_front(off, sb, db, ei_d, dn, ef, smDen, smEe):
            di1 = pltpu.async_copy(src_hbm.at[off // K], sb, sem7)
            di2 = pltpu.async_copy(dst_hbm.at[off // K], db, sem8)
            dee = pltpu.async_copy(ee_hbm.at[pl.ds(off * NH, KB)], ef, smEe)
            di1.wait()
            di2.wait()

            if NH == 8:
                def bidx(i):
                    lane = pr + i * 2
                    dv = plsc.load_gather(db, [lane >> 6, lane & 63])
                    ei_d[pl.ds(i * 16, 16)] = dv * 8 + pc
                nb = K // 2
            else:
                def bidx(i):
                    lane = iota + i * 16
                    dv = plsc.load_gather(db, [lane >> 6, lane & 63])
                    ei_d[pl.ds(i * 16, 16)] = dv * 8
                nb = K // 16
            plsc.parallel_loop(0, nb, unroll=4)(bidx)

            ds = []
            for j in range(NT):
                ds.append(pltpu.async_copy(
                    DENF.at[ei_d.at[pl.ds(j * K, K)]],
                    dn.at[pl.ds(j * K, K)], smDen))
            return ds, dee

        def p2_alpha(ds, dee, dn, ef):
            for d in ds:
                d.wait()
            dee.wait()

            def abody(i):
                ee = ef[pl.ds(i * 16, 16)]
                alphab[pl.ds(i * 16, 16)] = ee / (dn[pl.ds(i * 16, 16)]
                                                  + 1e-16)
            plsc.parallel_loop(0, KB // 16, unroll=4)(abody)

        def mk_mbody(buf, kofs):
            def mbody(k):
                k8 = (k + kofs) * NH
                for v in range(NV):
                    av = plsc.load_gather(
                        alphab,
                        [jnp.full((16,), k8 + (v * 16) // C, jnp.int32)])
                    hv = buf[k, pl.ds(v * 16, 16)]
                    buf[k, pl.ds(v * 16, 16)] = hv * av
            return mbody

        def p2_pair(j2, _):
            offA = s * E_TILE + j2 * 2 * K
            offB = offA + K
            dsA, deeA = p2_front(offA, srcb, dstb, eid, asr, eef, sem2, sem5)
            hA1 = pltpu.async_copy(h_hbm.at[srcb.at[0]], hrowsA, sem1)
            hA2 = pltpu.async_copy(h_hbm.at[srcb.at[1]], hrowsB, sem3)
            dsB, deeB = p2_front(offB, srcbB, dstbB, eidB, asrB, eefB,
                                 sem4, sem6)
            p2_alpha(dsA, deeA, asr, eef)
            hA1.wait()
            plsc.parallel_loop(0, K // 2, unroll=4)(mk_mbody(hrowsA, 0))
            sA1 = pltpu.async_copy(hrowsA, OUT.at[dstb.at[0]], sem9,
                                   add=True)
            hA2.wait()
            plsc.parallel_loop(0, K // 2, unroll=4)(mk_mbody(hrowsB, K // 2))
            sA2 = pltpu.async_copy(hrowsB, OUT.at[dstb.at[1]], sem10,
                                   add=True)
            p2_alpha(dsB, deeB, asrB, eefB)
            sA1.wait()
            hB1 = pltpu.async_copy(h_hbm.at[srcbB.at[0]], hrowsA, sem1)
            sA2.wait()
            hB2 = pltpu.async_copy(h_hbm.at[srcbB.at[1]], hrowsB, sem3)
            hB1.wait()
            plsc.parallel_loop(0, K // 2, unroll=4)(mk_mbody(hrowsA, 0))
            sB1 = pltpu.async_copy(hrowsA, OUT.at[dstbB.at[0]], sem9,
                                   add=True)
            hB2.wait()
            plsc.parallel_loop(0, K // 2, unroll=4)(mk_mbody(hrowsB, K // 2))
            sB2 = pltpu.async_copy(hrowsB, OUT.at[dstbB.at[1]], sem10,
                                   add=True)
            sB1.wait()
            sB2.wait()
            return 0
        lax.fori_loop(0, NCH // 2, p2_pair, 0)
        plsc.subcore_barrier()

        for o, ln in _chunks(ROWS_PER_TILE, K):
            r = row0 + o
            pltpu.sync_copy(OUT.at[pl.ds(r, ln)], out_hbm.at[pl.ds(r, ln)])

    return sc_gat


_sc_gat1 = _make_sc_gat(8, C1, 8)
_sc_gat2 = _make_sc_gat(4, C2, 1)


def _att_mat(att, dc):
    H, _ = att.shape
    S = att[:, :, None] * jnp.eye(H, dtype=jnp.float32)[:, None, :]
    S = S.reshape(dc, H)
    return jnp.pad(S, ((0, 0), (0, 8 - H)))


def kernel(x, edge_index, W1, att_src1, att_dst1, b1, W2, att_src2, att_dst2,
           b2):
    xp = jnp.pad(x, ((0, N_PAD - N_NODES), (0, 0)))
    loops = jnp.arange(N_NODES, dtype=jnp.int32)
    fill = jnp.full((E_PAD - N_E,), N_NODES, jnp.int32)
    src = jnp.concatenate([edge_index[0].astype(jnp.int32), loops, fill])
    dst = jnp.concatenate([edge_index[1].astype(jnp.int32), loops, fill])
    src2d = src.reshape(E_PAD // K, 2, K // 2)
    dst2d = dst.reshape(E_PAD // K, 2, K // 2)

    Ss1 = _att_mat(att_src1, D_FEAT)
    Sd1 = _att_mat(att_dst1, D_FEAT)
    Ss2 = _att_mat(att_src2, C2)
    Sd2 = _att_mat(att_dst2, C2)

    h1, AS1, AD1 = _tc_proj(xp, W1, Ss1, Sd1)
    p1, _ = _sc_gat1(h1, AS1.reshape(-1), AD1.reshape(-1), src2d, dst2d)
    h2, AS2, AD2 = _tc_mid(p1, b1.reshape(1, -1), W2, Ss2, Sd2)
    p2, _ = _sc_gat2(h2, AS2.reshape(-1), AD2.reshape(-1), src2d, dst2d)
    b2p = jnp.pad(b2, (0, 128 - C2)).reshape(1, 128)
    out = _tc_final(p2, b2p)
    return out[:N_NODES, :C2]
